# Initial kernel scaffold; baseline (speedup 1.0000x reference)
import jax
import jax.numpy as jnp
from jax.experimental import pallas as pl


def kernel(t, pos, h, l, W_emb, b_emb, W_t, W_lin, W_vin, We_s, We_d, We_f, be, Wn, bn, ln_g, ln_b, W_vout, W_lout, node_index, edge_node_index):
    raise NotImplementedError("write your pallas kernel here")


# speedup vs baseline: 1.6677x; 1.6677x over previous
import math

import jax
import jax.numpy as jnp
from jax import lax
from jax.experimental import pallas as pl
from jax.experimental.pallas import tpu as pltpu
from jax.experimental.pallas import tpu_sc as plsc

F32 = jnp.float32
G = 256
N = 10000
E = 160000
H = 128
NL = 4
TEPS = 1e-3
B0, B1 = 0.1, 20.0
LOGRATIO = math.log(1.0 / 0.01)
SMIN = 0.01
LOG1E4 = math.log(10000.0)
TWO_PI = 2.0 * math.pi

NB = 1000
NB2 = 2000
BE = 2000

NC = 2
NS = 16
NW = NC * NS
EB = 128
NB_E = E // EB
NB_W = -(-NB_E // NW)
NPAD = 10240
AGG_HALF = NPAD // 2
AGG_ROWS = AGG_HALF + EB
ZPS = AGG_ROWS // NS
CPS = AGG_HALF // NS
ZR = 128


def _silu(v):
    return v / (1.0 + jnp.exp(-v))


def _onehot(nid_blk):
    ni = nid_blk[:, 0:1]
    cols = lax.broadcasted_iota(jnp.int32, (nid_blk.shape[0], G), 1)
    return (ni == cols).astype(F32)


def _dotT(a, b):
    return lax.dot_general(a, b, (((0,), (0,)), ((), ())),
                           preferred_element_type=F32)


def _tgmat(t_mat):
    return TEPS + (1.0 - 2.0 * TEPS) * t_mat


def _node_noise(O, tg_mat, pos_blk, epsr_blk):
    t_node = jnp.dot(O, tg_mat, preferred_element_type=F32)
    sig = SMIN * jnp.exp(t_node * LOGRATIO)
    ft = pos_blk + sig * epsr_blk
    ft = ft - jnp.floor(ft)
    d0 = ft - pos_blk
    rt = jnp.where(d0 > 0.5, d0 - 1.0, jnp.where(d0 < -0.5, d0 + 1.0, d0))
    u = -rt / (sig * sig)
    return ft, u


def _prep_a(nid_ref, t_ref, pos_ref, epsr_ref, epsv_ref,
            cnt_ref, sv_ref, su_ref, fpad_ref):
    pid = pl.program_id(0)
    O = _onehot(nid_ref[...])
    tg_mat = _tgmat(t_ref[...])
    ft, u = _node_noise(O, tg_mat, pos_ref[...], epsr_ref[...])
    fpad_ref[...] = ft
    ones = jnp.ones((NB, H), F32)

    @pl.when(pid == 0)
    def _():
        cnt_ref[...] = jnp.zeros((G, H), F32)
        sv_ref[...] = jnp.zeros((G, H), F32)
        su_ref[...] = jnp.zeros((G, H), F32)

    cnt_ref[...] += _dotT(O, ones)
    sv_ref[...] += _dotT(O, epsv_ref[...])
    su_ref[...] += _dotT(O, u)


def _prep_b(nid_ref, t_ref, pos_ref, epsr_ref, epsv_ref, h_ref,
            wemb_ref, wt_ref, wlin_ref, wvin_ref, l9_ref, epsl9_ref,
            bemb_ref, cnt_ref, sv_ref, su_ref, x0_ref, tv_ref):
    O = _onehot(nid_ref[...])
    tg_mat = _tgmat(t_ref[...])
    _, u = _node_noise(O, tg_mat, pos_ref[...], epsr_ref[...])
    cntc = jnp.maximum(cnt_ref[...], 1.0)
    mean_v = sv_ref[...] / cntc
    mean_u = su_ref[...] / cntc
    v_t = epsv_ref[...] - jnp.dot(O, mean_v, preferred_element_type=F32)
    tv_ref[...] = u - jnp.dot(O, mean_u, preferred_element_type=F32) - v_t
    idx = lax.broadcasted_iota(jnp.int32, (G, H), 1)
    k = jnp.where(idx < 64, idx, idx - 64).astype(F32)
    fr = jnp.exp(-LOG1E4 * k / 63.0)
    a = tg_mat * fr
    temb_in = jnp.where(idx < 64, jnp.sin(a), jnp.cos(a))
    log_ab = -0.25 * tg_mat * tg_mat * (B1 - B0) - 0.5 * tg_mat * B0
    ab = jnp.exp(log_ab)
    ltp = jnp.sqrt(ab) * l9_ref[...] + jnp.sqrt(1.0 - ab) * epsl9_ref[...]
    gnode = (jnp.dot(temb_in, wt_ref[...], preferred_element_type=F32)
             + jnp.dot(ltp, wlin_ref[...], preferred_element_type=F32)
             + bemb_ref[...])
    x0_ref[...] = (jnp.dot(h_ref[...], wemb_ref[...], preferred_element_type=F32)
                   + jnp.dot(O, gnode, preferred_element_type=F32)
                   + jnp.dot(v_t, wvin_ref[...], preferred_element_type=F32))


def _fourier(dd_ref, wf_ref, o0_ref, o1_ref, o2_ref, o3_ref):
    dd = dd_ref[...]
    kv = (lax.broadcasted_iota(jnp.int32, (BE, H), 1) + 1).astype(F32)
    parts = []
    for j in range(3):
        aj = TWO_PI * dd[:, j:j + 1] * kv
        parts.append(jnp.sin(aj))
        parts.append(jnp.cos(aj))
    ff = jnp.concatenate(parts, axis=1)
    outs = (o0_ref, o1_ref, o2_ref, o3_ref)
    for i in range(NL):
        outs[i][...] = jnp.dot(ff, wf_ref[i], preferred_element_type=F32)


def _node_mm(x_ref, ws_ref, wd_ref, be_ref, xs_ref, xd_ref):
    x = x_ref[...]
    xs_ref[...] = jnp.dot(x, ws_ref[...], preferred_element_type=F32)
    xd_ref[...] = (jnp.dot(x, wd_ref[...], preferred_element_type=F32)
                   + be_ref[...][0:1, :])


def _update(x_ref, a0_ref, a1_ref, w1_ref, w2_ref, bn_ref, g_ref, b_ref,
            y_ref):
    x = x_ref[...]
    agg = a0_ref[0] + a1_ref[0]
    u = (jnp.dot(x, w1_ref[...], preferred_element_type=F32)
         + jnp.dot(agg, w2_ref[...], preferred_element_type=F32)
         + bn_ref[...][0:1, :])
    y = x + _silu(u)
    m = jnp.mean(y, axis=1, keepdims=True)
    yc = y - m
    var = jnp.mean(yc * yc, axis=1, keepdims=True)
    y_ref[...] = (yc * lax.rsqrt(var + 1e-5) * g_ref[...][0:1, :]
                  + b_ref[...][0:1, :])


def _head_a(x_ref, nid_ref, wv_ref, pooled_ref, sy_ref):
    pid = pl.program_id(0)
    O = _onehot(nid_ref[...])
    x = x_ref[...]
    y = jnp.dot(x, wv_ref[...], preferred_element_type=F32)

    @pl.when(pid == 0)
    def _():
        pooled_ref[...] = jnp.zeros((G, H), F32)
        sy_ref[...] = jnp.zeros((G, H), F32)

    pooled_ref[...] += _dotT(O, x)
    sy_ref[...] += _dotT(O, y)


def _head_b(x_ref, nid_ref, tv_ref, wv_ref, cnt_ref, sy_ref, pooled_ref,
            wl_ref, tl_ref, loss_ref):
    pid = pl.program_id(0)
    O = _onehot(nid_ref[...])
    x = x_ref[...]
    y = jnp.dot(x, wv_ref[...], preferred_element_type=F32)
    cntc = jnp.maximum(cnt_ref[...], 1.0)
    meany = sy_ref[...] / cntc
    pv = y - jnp.dot(O, meany, preferred_element_type=F32)
    d = pv - tv_ref[...]
    s = (jnp.sum(d * d) / (3.0 * N)).reshape(1, 1)

    @pl.when(pid == 0)
    def _():
        loss_ref[...] = jnp.zeros((1, 1), F32)

    loss_ref[...] += s

    @pl.when(pid == (N // NB) - 1)
    def _():
        pm = pooled_ref[...] / cntc
        plh = jnp.dot(pm, wl_ref[...], preferred_element_type=F32)
        dl = plh - tl_ref[...]
        loss_ref[...] += (jnp.sum(dl * dl) / (9.0 * G)).reshape(1, 1)


def _dd_body(fpad_hbm, src_hbm, dst_hbm, dd_hbm, idx_s, idx_d, fs, fd, dv,
             sem):
    c = lax.axis_index("c")
    s = lax.axis_index("s")
    wid = s * NC + c

    def blk(bi, carry):
        b = wid + bi * NW

        @pl.when(b < NB_E)
        def _():
            e0 = b * EB
            pltpu.sync_copy(src_hbm.at[pl.ds(e0, EB)], idx_s)
            pltpu.sync_copy(dst_hbm.at[pl.ds(e0, EB)], idx_d)
            pltpu.async_copy(fpad_hbm.at[idx_s], fs, sem).wait()
            pltpu.async_copy(fpad_hbm.at[idx_d], fd, sem).wait()

            def row(r, carry2):
                sl = pl.ds(0, 16)
                d = fs[r, sl] - fd[r, sl]
                d = jnp.where(d > 0.5, d - 1.0,
                              jnp.where(d < -0.5, d + 1.0, d))
                dv[r, sl] = d
                return carry2

            lax.fori_loop(0, EB, row, 0)
            pltpu.sync_copy(dv, dd_hbm.at[pl.ds(e0, EB)])

        return carry

    lax.fori_loop(0, NB_W, blk, 0)


def _edge_body(xs_hbm, xd_hbm, ffw_hbm, src_hbm, dst_hbm, out_hbm,
               idx_s, idx_d, gs, gd, fw, zb, agg_sh, sem):
    c = lax.axis_index("c")
    s = lax.axis_index("s")
    wid = s * NC + c
    zvec = jnp.zeros((16,), F32)

    def zrow(r, carry):
        for k2 in range(8):
            zb[r, pl.ds(k2 * 16, 16)] = zvec
        return carry

    lax.fori_loop(0, ZR, zrow, 0)

    for phase in range(2):
        base = phase * AGG_HALF
        pltpu.sync_copy(zb, agg_sh.at[pl.ds(s * ZPS, ZR)])
        pltpu.sync_copy(zb, agg_sh.at[pl.ds(s * ZPS + ZR, ZR)])
        pltpu.sync_copy(zb.at[pl.ds(0, ZPS - 2 * ZR)],
                        agg_sh.at[pl.ds(s * ZPS + 2 * ZR, ZPS - 2 * ZR)])
        plsc.subcore_barrier()

        def blk(bi, carry):
            b = wid + bi * NW

            @pl.when(b < NB_E)
            def _():
                e0 = b * EB
                pltpu.sync_copy(src_hbm.at[pl.ds(e0, EB)], idx_s)
                pltpu.sync_copy(dst_hbm.at[pl.ds(e0, EB)], idx_d)
                pltpu.async_copy(xs_hbm.at[idx_s], gs, sem).wait()
                pltpu.async_copy(xd_hbm.at[idx_d], gd, sem).wait()
                pltpu.sync_copy(ffw_hbm.at[pl.ds(e0, EB)], fw)

                def row(r, carry2):
                    for k2 in range(8):
                        sl = pl.ds(k2 * 16, 16)
                        v = gs[r, sl] + gd[r, sl] + fw[r, sl]
                        gs[r, sl] = v / (1.0 + jnp.exp(-v))
                    return carry2

                lax.fori_loop(0, EB, row, 0)

                def adj(k2, carry3):
                    sl = pl.ds(k2 * 16, 16)
                    tl = idx_d[sl] - base
                    ok = (tl >= 0) & (tl < AGG_HALF)
                    idx_d[sl] = jnp.where(ok, tl, AGG_HALF)
                    return carry3

                lax.fori_loop(0, 8, adj, 0)
                pltpu.sync_copy(gs, agg_sh.at[idx_d], add=True)

            return carry

        lax.fori_loop(0, NB_W, blk, 0)
        plsc.subcore_barrier()
        pltpu.sync_copy(agg_sh.at[pl.ds(s * CPS, CPS)],
                        out_hbm.at[c, pl.ds(base + s * CPS, CPS)])
        plsc.subcore_barrier()


import functools


@functools.lru_cache(maxsize=None)
def _sc_mesh():
    return plsc.VectorSubcoreMesh(core_axis_name="c", subcore_axis_name="s",
                                  num_cores=NC, num_subcores=NS)


@functools.lru_cache(maxsize=None)
def _dd_kernel():
    return pl.kernel(
        _dd_body,
        out_type=jax.ShapeDtypeStruct((E, H), F32),
        mesh=_sc_mesh(),
        scratch_types=[
            pltpu.VMEM((EB,), jnp.int32),
            pltpu.VMEM((EB,), jnp.int32),
            pltpu.VMEM((EB, H), F32),
            pltpu.VMEM((EB, H), F32),
            pltpu.VMEM((EB, H), F32),
            pltpu.SemaphoreType.DMA,
        ],
    )


@functools.lru_cache(maxsize=None)
def _edge_kernel():
    return pl.kernel(
        _edge_body,
        out_type=jax.ShapeDtypeStruct((NC, NPAD, H), F32),
        mesh=_sc_mesh(),
        scratch_types=[
            pltpu.VMEM((EB,), jnp.int32),
            pltpu.VMEM((EB,), jnp.int32),
            pltpu.VMEM((EB, H), F32),
            pltpu.VMEM((EB, H), F32),
            pltpu.VMEM((EB, H), F32),
            pltpu.VMEM((ZR, H), F32),
            pltpu.VMEM_SHARED((AGG_ROWS, H), F32),
            pltpu.SemaphoreType.DMA,
        ],
    )


def _dd_call(fpad, src, dst):
    return _dd_kernel()(fpad, src, dst)


def _edge_call(xs, xd, ffw, src, dst):
    return _edge_kernel()(xs, xd, ffw, src, dst)


def kernel(t, pos, h, l, W_emb, b_emb, W_t, W_lin, W_vin, We_s, We_d, We_f,
           be, Wn, bn, ln_g, ln_b, W_vout, W_lout, node_index,
           edge_node_index):
    nkey = jax.random.key(1)
    eps_l = jax.random.normal(jax.random.fold_in(nkey, 0), l.shape, dtype=F32)
    eps_r = jax.random.normal(jax.random.fold_in(nkey, 1), pos.shape, dtype=F32)
    eps_v = jax.random.normal(jax.random.fold_in(nkey, 2), pos.shape, dtype=F32)

    padc = lambda a: jnp.pad(a, ((0, 0), (0, H - a.shape[1])))
    padr = lambda a: jnp.pad(a, ((0, H - a.shape[0]), (0, 0)))
    pos_p = padc(pos)
    epsr_p = padc(eps_r)
    epsv_p = padc(eps_v)
    h_p = padc(h)
    wemb_p = padr(W_emb)
    wlin_p = padr(W_lin)
    wvin_p = padr(W_vin)
    wvout_p = padc(W_vout)
    wlout_p = padc(W_lout)
    l9p = padc(l.reshape(G, 9))
    epsl9p = padc(eps_l.reshape(G, 9))
    t_mat = jnp.broadcast_to(t[:, None], (G, H))
    nid_b = jnp.broadcast_to(node_index[:, None], (N, H))
    bemb_m = jnp.broadcast_to(b_emb[None, :], (G, H))
    src = edge_node_index[0]
    dst = edge_node_index[1]

    nblk = pl.BlockSpec((NB, H), lambda i: (i, 0))
    gblk = pl.BlockSpec((G, H), lambda i: (0, 0))
    wblk = pl.BlockSpec((H, H), lambda i: (0, 0))

    cnt, sv, su, fpad = pl.pallas_call(
        _prep_a,
        grid=(N // NB,),
        in_specs=[nblk, gblk, nblk, nblk, nblk],
        out_specs=[gblk, gblk, gblk, nblk],
        out_shape=[jax.ShapeDtypeStruct((G, H), F32)] * 3
        + [jax.ShapeDtypeStruct((N, H), F32)],
    )(nid_b, t_mat, pos_p, epsr_p, epsv_p)

    x0, tv = pl.pallas_call(
        _prep_b,
        grid=(N // NB,),
        in_specs=[nblk, gblk, nblk, nblk, nblk, nblk,
                  wblk, wblk, wblk, wblk, gblk, gblk, gblk,
                  gblk, gblk, gblk],
        out_specs=[nblk, nblk],
        out_shape=[jax.ShapeDtypeStruct((N, H), F32)] * 2,
    )(nid_b, t_mat, pos_p, epsr_p, epsv_p, h_p, wemb_p, W_t, wlin_p,
      wvin_p, l9p, epsl9p, bemb_m, cnt, sv, su)

    dd = _dd_call(fpad, src, dst)

    eblk = pl.BlockSpec((BE, H), lambda i: (i, 0))
    ffw = pl.pallas_call(
        _fourier,
        grid=(E // BE,),
        in_specs=[eblk,
                  pl.BlockSpec((NL, 768, H), lambda i: (0, 0, 0))],
        out_specs=[eblk, eblk, eblk, eblk],
        out_shape=[jax.ShapeDtypeStruct((E, H), F32)] * NL,
    )(dd, We_f)

    n2blk = pl.BlockSpec((NB2, H), lambda i: (i, 0))
    rowblk = pl.BlockSpec((8, H), lambda i: (0, 0))
    x = x0
    for i in range(NL):
        be_b = jnp.broadcast_to(be[i][None, :], (8, H))
        bn_b = jnp.broadcast_to(bn[i][None, :], (8, H))
        g_b = jnp.broadcast_to(ln_g[i][None, :], (8, H))
        b_b = jnp.broadcast_to(ln_b[i][None, :], (8, H))
        xs, xd = pl.pallas_call(
            _node_mm,
            grid=(N // NB2,),
            in_specs=[n2blk, wblk, wblk, rowblk],
            out_specs=[n2blk, n2blk],
            out_shape=[jax.ShapeDtypeStruct((N, H), F32)] * 2,
        )(x, We_s[i], We_d[i], be_b)

        aggp = _edge_call(xs, xd, ffw[i], src, dst)

        x = pl.pallas_call(
            _update,
            grid=(N // NB2,),
            in_specs=[n2blk,
                      pl.BlockSpec((1, NB2, H), lambda i: (0, i, 0)),
                      pl.BlockSpec((1, NB2, H), lambda i: (1, i, 0)),
                      wblk, wblk, rowblk, rowblk, rowblk],
            out_specs=n2blk,
            out_shape=jax.ShapeDtypeStruct((N, H), F32),
        )(x, aggp, aggp, Wn[i, :H, :], Wn[i, H:, :], bn_b, g_b, b_b)

    pooled, sy = pl.pallas_call(
        _head_a,
        grid=(N // NB,),
        in_specs=[nblk, nblk, wblk],
        out_specs=[gblk, gblk],
        out_shape=[jax.ShapeDtypeStruct((G, H), F32)] * 2,
    )(x, nid_b, wvout_p)

    loss = pl.pallas_call(
        _head_b,
        grid=(N // NB,),
        in_specs=[nblk, nblk, nblk, wblk, gblk, gblk, gblk, wblk, gblk],
        out_specs=pl.BlockSpec((1, 1), lambda i: (0, 0)),
        out_shape=jax.ShapeDtypeStruct((1, 1), F32),
    )(x, nid_b, tv, wvout_p, cnt, sy, pooled, wlout_p, epsl9p)

    return loss[0, 0]


# speedup vs baseline: 2.1342x; 1.2798x over previous
import math

import jax
import jax.numpy as jnp
from jax import lax
from jax.experimental import pallas as pl
from jax.experimental.pallas import tpu as pltpu
from jax.experimental.pallas import tpu_sc as plsc

F32 = jnp.float32
G = 256
N = 10000
E = 160000
H = 128
NL = 4
TEPS = 1e-3
B0, B1 = 0.1, 20.0
LOGRATIO = math.log(1.0 / 0.01)
SMIN = 0.01
LOG1E4 = math.log(10000.0)
TWO_PI = 2.0 * math.pi

NB = 1000
NB2 = 2000
BE = 2000

NC = 2
NS = 16
NW = NC * NS
EB = 128
NB_E = E // EB
NB_W = -(-NB_E // NW)
NPAD = 10240
AGG_HALF = NPAD // 2
AGG_ROWS = AGG_HALF + EB
ZPS = AGG_ROWS // NS
CPS = AGG_HALF // NS
ZR = 128


def _silu(v):
    return v / (1.0 + jnp.exp(-v))


def _onehot(nid_blk):
    ni = nid_blk[:, 0:1]
    cols = lax.broadcasted_iota(jnp.int32, (nid_blk.shape[0], G), 1)
    return (ni == cols).astype(F32)


def _dotT(a, b):
    return lax.dot_general(a, b, (((0,), (0,)), ((), ())),
                           preferred_element_type=F32)


def _tgmat(t_mat):
    return TEPS + (1.0 - 2.0 * TEPS) * t_mat


def _node_noise(O, tg_mat, pos_blk, epsr_blk):
    t_node = jnp.dot(O, tg_mat, preferred_element_type=F32)
    sig = SMIN * jnp.exp(t_node * LOGRATIO)
    ft = pos_blk + sig * epsr_blk
    ft = ft - jnp.floor(ft)
    d0 = ft - pos_blk
    rt = jnp.where(d0 > 0.5, d0 - 1.0, jnp.where(d0 < -0.5, d0 + 1.0, d0))
    u = -rt / (sig * sig)
    return ft, u


def _prep_a(nid_ref, t_ref, pos_ref, epsr_ref, epsv_ref,
            cnt_ref, sv_ref, su_ref, fpad_ref):
    pid = pl.program_id(0)
    O = _onehot(nid_ref[...])
    tg_mat = _tgmat(t_ref[...])
    ft, u = _node_noise(O, tg_mat, pos_ref[...], epsr_ref[...])
    fpad_ref[...] = ft
    ones = jnp.ones((NB, H), F32)

    @pl.when(pid == 0)
    def _():
        cnt_ref[...] = jnp.zeros((G, H), F32)
        sv_ref[...] = jnp.zeros((G, H), F32)
        su_ref[...] = jnp.zeros((G, H), F32)

    cnt_ref[...] += _dotT(O, ones)
    sv_ref[...] += _dotT(O, epsv_ref[...])
    su_ref[...] += _dotT(O, u)


def _prep_b(nid_ref, t_ref, pos_ref, epsr_ref, epsv_ref, h_ref,
            wemb_ref, wt_ref, wlin_ref, wvin_ref, l9_ref, epsl9_ref,
            bemb_ref, cnt_ref, sv_ref, su_ref, x0_ref, tv_ref):
    O = _onehot(nid_ref[...])
    tg_mat = _tgmat(t_ref[...])
    _, u = _node_noise(O, tg_mat, pos_ref[...], epsr_ref[...])
    cntc = jnp.maximum(cnt_ref[...], 1.0)
    mean_v = sv_ref[...] / cntc
    mean_u = su_ref[...] / cntc
    v_t = epsv_ref[...] - jnp.dot(O, mean_v, preferred_element_type=F32)
    tv_ref[...] = u - jnp.dot(O, mean_u, preferred_element_type=F32) - v_t
    idx = lax.broadcasted_iota(jnp.int32, (G, H), 1)
    k = jnp.where(idx < 64, idx, idx - 64).astype(F32)
    fr = jnp.exp(-LOG1E4 * k / 63.0)
    a = tg_mat * fr
    temb_in = jnp.where(idx < 64, jnp.sin(a), jnp.cos(a))
    log_ab = -0.25 * tg_mat * tg_mat * (B1 - B0) - 0.5 * tg_mat * B0
    ab = jnp.exp(log_ab)
    ltp = jnp.sqrt(ab) * l9_ref[...] + jnp.sqrt(1.0 - ab) * epsl9_ref[...]
    gnode = (jnp.dot(temb_in, wt_ref[...], preferred_element_type=F32)
             + jnp.dot(ltp, wlin_ref[...], preferred_element_type=F32)
             + bemb_ref[...])
    x0_ref[...] = (jnp.dot(h_ref[...], wemb_ref[...], preferred_element_type=F32)
                   + jnp.dot(O, gnode, preferred_element_type=F32)
                   + jnp.dot(v_t, wvin_ref[...], preferred_element_type=F32))


def _fourier(dd_ref, wf_ref, o0_ref, o1_ref, o2_ref, o3_ref):
    dd = dd_ref[...]
    kv = (lax.broadcasted_iota(jnp.int32, (BE, H), 1) + 1).astype(F32)
    parts = []
    for j in range(3):
        aj = TWO_PI * dd[:, j:j + 1] * kv
        parts.append(jnp.sin(aj))
        parts.append(jnp.cos(aj))
    ff = jnp.concatenate(parts, axis=1)
    outs = (o0_ref, o1_ref, o2_ref, o3_ref)
    for i in range(NL):
        outs[i][...] = jnp.dot(ff, wf_ref[i], preferred_element_type=F32)


def _node_mm(x_ref, ws_ref, wd_ref, be_ref, xs_ref, xd_ref):
    x = x_ref[...]
    xs_ref[...] = jnp.dot(x, ws_ref[...], preferred_element_type=F32)
    xd_ref[...] = (jnp.dot(x, wd_ref[...], preferred_element_type=F32)
                   + be_ref[...][0:1, :])


def _update(x_ref, a0_ref, a1_ref, w1_ref, w2_ref, bn_ref, g_ref, b_ref,
            y_ref):
    x = x_ref[...]
    agg = a0_ref[0] + a1_ref[0]
    u = (jnp.dot(x, w1_ref[...], preferred_element_type=F32)
         + jnp.dot(agg, w2_ref[...], preferred_element_type=F32)
         + bn_ref[...][0:1, :])
    y = x + _silu(u)
    m = jnp.mean(y, axis=1, keepdims=True)
    yc = y - m
    var = jnp.mean(yc * yc, axis=1, keepdims=True)
    y_ref[...] = (yc * lax.rsqrt(var + 1e-5) * g_ref[...][0:1, :]
                  + b_ref[...][0:1, :])


def _head_a(x_ref, nid_ref, wv_ref, pooled_ref, sy_ref):
    pid = pl.program_id(0)
    O = _onehot(nid_ref[...])
    x = x_ref[...]
    y = jnp.dot(x, wv_ref[...], preferred_element_type=F32)

    @pl.when(pid == 0)
    def _():
        pooled_ref[...] = jnp.zeros((G, H), F32)
        sy_ref[...] = jnp.zeros((G, H), F32)

    pooled_ref[...] += _dotT(O, x)
    sy_ref[...] += _dotT(O, y)


def _head_b(x_ref, nid_ref, tv_ref, wv_ref, cnt_ref, sy_ref, pooled_ref,
            wl_ref, tl_ref, loss_ref):
    pid = pl.program_id(0)
    O = _onehot(nid_ref[...])
    x = x_ref[...]
    y = jnp.dot(x, wv_ref[...], preferred_element_type=F32)
    cntc = jnp.maximum(cnt_ref[...], 1.0)
    meany = sy_ref[...] / cntc
    pv = y - jnp.dot(O, meany, preferred_element_type=F32)
    d = pv - tv_ref[...]
    s = (jnp.sum(d * d) / (3.0 * N)).reshape(1, 1)

    @pl.when(pid == 0)
    def _():
        loss_ref[...] = jnp.zeros((1, 1), F32)

    loss_ref[...] += s

    @pl.when(pid == (N // NB) - 1)
    def _():
        pm = pooled_ref[...] / cntc
        plh = jnp.dot(pm, wl_ref[...], preferred_element_type=F32)
        dl = plh - tl_ref[...]
        loss_ref[...] += (jnp.sum(dl * dl) / (9.0 * G)).reshape(1, 1)


def _dd_body(fpad_hbm, src_hbm, dst_hbm, dd_hbm, idx_s, idx_d, fs, fd, dv,
             sem):
    c = lax.axis_index("c")
    s = lax.axis_index("s")
    wid = s * NC + c

    def blk(bi, carry):
        b = wid + bi * NW

        @pl.when(b < NB_E)
        def _():
            e0 = b * EB
            pltpu.sync_copy(src_hbm.at[pl.ds(e0, EB)], idx_s)
            pltpu.sync_copy(dst_hbm.at[pl.ds(e0, EB)], idx_d)
            pltpu.async_copy(fpad_hbm.at[idx_s], fs, sem).wait()
            pltpu.async_copy(fpad_hbm.at[idx_d], fd, sem).wait()

            def row(r, carry2):
                sl = pl.ds(0, 16)
                d = fs[r, sl] - fd[r, sl]
                d = jnp.where(d > 0.5, d - 1.0,
                              jnp.where(d < -0.5, d + 1.0, d))
                dv[r, sl] = d
                return carry2

            lax.fori_loop(0, EB, row, 0)
            pltpu.sync_copy(dv, dd_hbm.at[pl.ds(e0, EB)])

        return carry

    lax.fori_loop(0, NB_W, blk, 0)


def _edge_body(xs_hbm, xd_hbm, ffw_hbm, src_hbm, dst_hbm, out_hbm, m_hbm,
               idx_s, idx_d, gs, gd, fw, zb, agg_sh, sem):
    c = lax.axis_index("c")
    s = lax.axis_index("s")
    wid = s * NC + c
    zvec = jnp.zeros((16,), F32)

    def zrow(r, carry):
        for k2 in range(8):
            zb[r, pl.ds(k2 * 16, 16)] = zvec
        return carry

    lax.fori_loop(0, ZR, zrow, 0)

    def zero_agg():
        pltpu.sync_copy(zb, agg_sh.at[pl.ds(s * ZPS, ZR)])
        pltpu.sync_copy(zb, agg_sh.at[pl.ds(s * ZPS + ZR, ZR)])
        pltpu.sync_copy(zb.at[pl.ds(0, ZPS - 2 * ZR)],
                        agg_sh.at[pl.ds(s * ZPS + 2 * ZR, ZPS - 2 * ZR)])
        plsc.subcore_barrier()

    def scatter(base):
        def adj(k2, carry3):
            sl = pl.ds(k2 * 16, 16)
            tl = idx_d[sl] - base
            ok = (tl >= 0) & (tl < AGG_HALF)
            idx_d[sl] = jnp.where(ok, tl, AGG_HALF)
            return carry3

        lax.fori_loop(0, 8, adj, 0)
        pltpu.sync_copy(gs, agg_sh.at[idx_d], add=True)

    def dump_agg(base):
        plsc.subcore_barrier()
        pltpu.sync_copy(agg_sh.at[pl.ds(s * CPS, CPS)],
                        out_hbm.at[c, pl.ds(base + s * CPS, CPS)])
        plsc.subcore_barrier()

    zero_agg()

    def blk0(bi, carry):
        b = wid + bi * NW

        @pl.when(b < NB_E)
        def _():
            e0 = b * EB
            pltpu.sync_copy(src_hbm.at[pl.ds(e0, EB)], idx_s)
            pltpu.sync_copy(dst_hbm.at[pl.ds(e0, EB)], idx_d)
            pltpu.async_copy(xs_hbm.at[idx_s], gs, sem).wait()
            pltpu.async_copy(xd_hbm.at[idx_d], gd, sem).wait()
            pltpu.sync_copy(ffw_hbm.at[pl.ds(e0, EB)], fw)

            @plsc.parallel_loop(0, EB, step=1, unroll=4)
            def row(r):
                for k2 in range(8):
                    sl = pl.ds(k2 * 16, 16)
                    v = gs[r, sl] + gd[r, sl] + fw[r, sl]
                    gs[r, sl] = v / (1.0 + jnp.exp(-v))

            pltpu.sync_copy(gs, m_hbm.at[pl.ds(e0, EB)])
            scatter(0)

        return carry

    lax.fori_loop(0, NB_W, blk0, 0)
    dump_agg(0)

    zero_agg()

    def blk1(bi, carry):
        b = wid + bi * NW

        @pl.when(b < NB_E)
        def _():
            e0 = b * EB
            pltpu.sync_copy(dst_hbm.at[pl.ds(e0, EB)], idx_d)
            pltpu.sync_copy(m_hbm.at[pl.ds(e0, EB)], gs)
            scatter(AGG_HALF)

        return carry

    lax.fori_loop(0, NB_W, blk1, 0)
    dump_agg(AGG_HALF)


import functools


@functools.lru_cache(maxsize=None)
def _sc_mesh():
    return plsc.VectorSubcoreMesh(core_axis_name="c", subcore_axis_name="s",
                                  num_cores=NC, num_subcores=NS)


@functools.lru_cache(maxsize=None)
def _dd_kernel():
    return pl.kernel(
        _dd_body,
        out_type=jax.ShapeDtypeStruct((E, H), F32),
        mesh=_sc_mesh(),
        scratch_types=[
            pltpu.VMEM((EB,), jnp.int32),
            pltpu.VMEM((EB,), jnp.int32),
            pltpu.VMEM((EB, H), F32),
            pltpu.VMEM((EB, H), F32),
            pltpu.VMEM((EB, H), F32),
            pltpu.SemaphoreType.DMA,
        ],
    )


@functools.lru_cache(maxsize=None)
def _edge_kernel():
    return pl.kernel(
        _edge_body,
        out_type=(jax.ShapeDtypeStruct((NC, NPAD, H), F32),
                  jax.ShapeDtypeStruct((E, H), F32)),
        mesh=_sc_mesh(),
        scratch_types=[
            pltpu.VMEM((EB,), jnp.int32),
            pltpu.VMEM((EB,), jnp.int32),
            pltpu.VMEM((EB, H), F32),
            pltpu.VMEM((EB, H), F32),
            pltpu.VMEM((EB, H), F32),
            pltpu.VMEM((ZR, H), F32),
            pltpu.VMEM_SHARED((AGG_ROWS, H), F32),
            pltpu.SemaphoreType.DMA,
        ],
    )


def _dd_call(fpad, src, dst):
    return _dd_kernel()(fpad, src, dst)


def _edge_call(xs, xd, ffw, src, dst):
    aggp, _ = _edge_kernel()(xs, xd, ffw, src, dst)
    return aggp


def kernel(t, pos, h, l, W_emb, b_emb, W_t, W_lin, W_vin, We_s, We_d, We_f,
           be, Wn, bn, ln_g, ln_b, W_vout, W_lout, node_index,
           edge_node_index):
    nkey = jax.random.key(1)
    eps_l = jax.random.normal(jax.random.fold_in(nkey, 0), l.shape, dtype=F32)
    eps_r = jax.random.normal(jax.random.fold_in(nkey, 1), pos.shape, dtype=F32)
    eps_v = jax.random.normal(jax.random.fold_in(nkey, 2), pos.shape, dtype=F32)

    padc = lambda a: jnp.pad(a, ((0, 0), (0, H - a.shape[1])))
    padr = lambda a: jnp.pad(a, ((0, H - a.shape[0]), (0, 0)))
    pos_p = padc(pos)
    epsr_p = padc(eps_r)
    epsv_p = padc(eps_v)
    h_p = padc(h)
    wemb_p = padr(W_emb)
    wlin_p = padr(W_lin)
    wvin_p = padr(W_vin)
    wvout_p = padc(W_vout)
    wlout_p = padc(W_lout)
    l9p = padc(l.reshape(G, 9))
    epsl9p = padc(eps_l.reshape(G, 9))
    t_mat = jnp.broadcast_to(t[:, None], (G, H))
    nid_b = jnp.broadcast_to(node_index[:, None], (N, H))
    bemb_m = jnp.broadcast_to(b_emb[None, :], (G, H))
    src = edge_node_index[0]
    dst = edge_node_index[1]

    nblk = pl.BlockSpec((NB, H), lambda i: (i, 0))
    gblk = pl.BlockSpec((G, H), lambda i: (0, 0))
    wblk = pl.BlockSpec((H, H), lambda i: (0, 0))

    cnt, sv, su, fpad = pl.pallas_call(
        _prep_a,
        grid=(N // NB,),
        in_specs=[nblk, gblk, nblk, nblk, nblk],
        out_specs=[gblk, gblk, gblk, nblk],
        out_shape=[jax.ShapeDtypeStruct((G, H), F32)] * 3
        + [jax.ShapeDtypeStruct((N, H), F32)],
    )(nid_b, t_mat, pos_p, epsr_p, epsv_p)

    x0, tv = pl.pallas_call(
        _prep_b,
        grid=(N // NB,),
        in_specs=[nblk, gblk, nblk, nblk, nblk, nblk,
                  wblk, wblk, wblk, wblk, gblk, gblk, gblk,
                  gblk, gblk, gblk],
        out_specs=[nblk, nblk],
        out_shape=[jax.ShapeDtypeStruct((N, H), F32)] * 2,
    )(nid_b, t_mat, pos_p, epsr_p, epsv_p, h_p, wemb_p, W_t, wlin_p,
      wvin_p, l9p, epsl9p, bemb_m, cnt, sv, su)

    dd = _dd_call(fpad, src, dst)

    eblk = pl.BlockSpec((BE, H), lambda i: (i, 0))
    ffw = pl.pallas_call(
        _fourier,
        grid=(E // BE,),
        in_specs=[eblk,
                  pl.BlockSpec((NL, 768, H), lambda i: (0, 0, 0))],
        out_specs=[eblk, eblk, eblk, eblk],
        out_shape=[jax.ShapeDtypeStruct((E, H), F32)] * NL,
    )(dd, We_f)

    n2blk = pl.BlockSpec((NB2, H), lambda i: (i, 0))
    rowblk = pl.BlockSpec((8, H), lambda i: (0, 0))
    x = x0
    for i in range(NL):
        be_b = jnp.broadcast_to(be[i][None, :], (8, H))
        bn_b = jnp.broadcast_to(bn[i][None, :], (8, H))
        g_b = jnp.broadcast_to(ln_g[i][None, :], (8, H))
        b_b = jnp.broadcast_to(ln_b[i][None, :], (8, H))
        xs, xd = pl.pallas_call(
            _node_mm,
            grid=(N // NB2,),
            in_specs=[n2blk, wblk, wblk, rowblk],
            out_specs=[n2blk, n2blk],
            out_shape=[jax.ShapeDtypeStruct((N, H), F32)] * 2,
        )(x, We_s[i], We_d[i], be_b)

        aggp = _edge_call(xs, xd, ffw[i], src, dst)

        x = pl.pallas_call(
            _update,
            grid=(N // NB2,),
            in_specs=[n2blk,
                      pl.BlockSpec((1, NB2, H), lambda i: (0, i, 0)),
                      pl.BlockSpec((1, NB2, H), lambda i: (1, i, 0)),
                      wblk, wblk, rowblk, rowblk, rowblk],
            out_specs=n2blk,
            out_shape=jax.ShapeDtypeStruct((N, H), F32),
        )(x, aggp, aggp, Wn[i, :H, :], Wn[i, H:, :], bn_b, g_b, b_b)

    pooled, sy = pl.pallas_call(
        _head_a,
        grid=(N // NB,),
        in_specs=[nblk, nblk, wblk],
        out_specs=[gblk, gblk],
        out_shape=[jax.ShapeDtypeStruct((G, H), F32)] * 2,
    )(x, nid_b, wvout_p)

    loss = pl.pallas_call(
        _head_b,
        grid=(N // NB,),
        in_specs=[nblk, nblk, nblk, wblk, gblk, gblk, gblk, wblk, gblk],
        out_specs=pl.BlockSpec((1, 1), lambda i: (0, 0)),
        out_shape=jax.ShapeDtypeStruct((1, 1), F32),
    )(x, nid_b, tv, wvout_p, cnt, sy, pooled, wlout_p, epsl9p)

    return loss[0, 0]


# speedup vs baseline: 2.3741x; 1.1124x over previous
import math

import jax
import jax.numpy as jnp
from jax import lax
from jax.experimental import pallas as pl
from jax.experimental.pallas import tpu as pltpu
from jax.experimental.pallas import tpu_sc as plsc

F32 = jnp.float32
G = 256
N = 10000
E = 160000
H = 128
NL = 4
TEPS = 1e-3
B0, B1 = 0.1, 20.0
LOGRATIO = math.log(1.0 / 0.01)
SMIN = 0.01
LOG1E4 = math.log(10000.0)
TWO_PI = 2.0 * math.pi

NB = 1000
NB2 = 2000
BE = 2000

NC = 2
NS = 16
NW = NC * NS
EB = 128
NB_E = E // EB
NB_W = -(-NB_E // NW)
NPAD = 10240
AGG_HALF = NPAD // 2
AGG_ROWS = AGG_HALF + EB
ZPS = AGG_ROWS // NS
CPS = AGG_HALF // NS
ZR = 128


def _silu(v):
    return v / (1.0 + jnp.exp(-v))


def _onehot(nid_blk):
    ni = nid_blk[:, 0:1]
    cols = lax.broadcasted_iota(jnp.int32, (nid_blk.shape[0], G), 1)
    return (ni == cols).astype(F32)


def _dotT(a, b):
    return lax.dot_general(a, b, (((0,), (0,)), ((), ())),
                           preferred_element_type=F32)


def _tgmat(t_mat):
    return TEPS + (1.0 - 2.0 * TEPS) * t_mat


def _node_noise(O, tg_mat, pos_blk, epsr_blk):
    t_node = jnp.dot(O, tg_mat, preferred_element_type=F32)
    sig = SMIN * jnp.exp(t_node * LOGRATIO)
    ft = pos_blk + sig * epsr_blk
    ft = ft - jnp.floor(ft)
    d0 = ft - pos_blk
    rt = jnp.where(d0 > 0.5, d0 - 1.0, jnp.where(d0 < -0.5, d0 + 1.0, d0))
    u = -rt / (sig * sig)
    return ft, u


def _prep_a(nid_ref, t_ref, pos_ref, epsr_ref, epsv_ref,
            cnt_ref, sv_ref, su_ref, fpad_ref):
    pid = pl.program_id(0)
    O = _onehot(nid_ref[...])
    tg_mat = _tgmat(t_ref[...])
    ft, u = _node_noise(O, tg_mat, pos_ref[...], epsr_ref[...])
    fpad_ref[...] = ft
    ones = jnp.ones((NB, H), F32)

    @pl.when(pid == 0)
    def _():
        cnt_ref[...] = jnp.zeros((G, H), F32)
        sv_ref[...] = jnp.zeros((G, H), F32)
        su_ref[...] = jnp.zeros((G, H), F32)

    cnt_ref[...] += _dotT(O, ones)
    sv_ref[...] += _dotT(O, epsv_ref[...])
    su_ref[...] += _dotT(O, u)


def _prep_b(nid_ref, t_ref, pos_ref, epsr_ref, epsv_ref, h_ref,
            wemb_ref, wt_ref, wlin_ref, wvin_ref, l9_ref, epsl9_ref,
            bemb_ref, cnt_ref, sv_ref, su_ref, x0_ref, tv_ref):
    O = _onehot(nid_ref[...])
    tg_mat = _tgmat(t_ref[...])
    _, u = _node_noise(O, tg_mat, pos_ref[...], epsr_ref[...])
    cntc = jnp.maximum(cnt_ref[...], 1.0)
    mean_v = sv_ref[...] / cntc
    mean_u = su_ref[...] / cntc
    v_t = epsv_ref[...] - jnp.dot(O, mean_v, preferred_element_type=F32)
    tv_ref[...] = u - jnp.dot(O, mean_u, preferred_element_type=F32) - v_t
    idx = lax.broadcasted_iota(jnp.int32, (G, H), 1)
    k = jnp.where(idx < 64, idx, idx - 64).astype(F32)
    fr = jnp.exp(-LOG1E4 * k / 63.0)
    a = tg_mat * fr
    temb_in = jnp.where(idx < 64, jnp.sin(a), jnp.cos(a))
    log_ab = -0.25 * tg_mat * tg_mat * (B1 - B0) - 0.5 * tg_mat * B0
    ab = jnp.exp(log_ab)
    ltp = jnp.sqrt(ab) * l9_ref[...] + jnp.sqrt(1.0 - ab) * epsl9_ref[...]
    gnode = (jnp.dot(temb_in, wt_ref[...], preferred_element_type=F32)
             + jnp.dot(ltp, wlin_ref[...], preferred_element_type=F32)
             + bemb_ref[...])
    x0_ref[...] = (jnp.dot(h_ref[...], wemb_ref[...], preferred_element_type=F32)
                   + jnp.dot(O, gnode, preferred_element_type=F32)
                   + jnp.dot(v_t, wvin_ref[...], preferred_element_type=F32))


def _fourier(dd_ref, wf_ref, o0_ref, o1_ref, o2_ref, o3_ref):
    dd = dd_ref[...]
    kv = (lax.broadcasted_iota(jnp.int32, (BE, H), 1) + 1).astype(F32)
    parts = []
    for j in range(3):
        aj = TWO_PI * dd[:, j:j + 1] * kv
        parts.append(jnp.sin(aj))
        parts.append(jnp.cos(aj))
    ff = jnp.concatenate(parts, axis=1).astype(jnp.bfloat16)
    outs = (o0_ref, o1_ref, o2_ref, o3_ref)
    for i in range(NL):
        outs[i][...] = jnp.dot(ff, wf_ref[i], preferred_element_type=F32)


def _node_mm(x_ref, ws_ref, wd_ref, be_ref, xs_ref, xd_ref):
    x = x_ref[...]
    xs_ref[...] = jnp.dot(x, ws_ref[...], preferred_element_type=F32)
    xd_ref[...] = (jnp.dot(x, wd_ref[...], preferred_element_type=F32)
                   + be_ref[...][0:1, :])


def _update(x_ref, a0_ref, a1_ref, w1_ref, w2_ref, bn_ref, g_ref, b_ref,
            y_ref):
    x = x_ref[...]
    agg = a0_ref[0] + a1_ref[0]
    u = (jnp.dot(x, w1_ref[...], preferred_element_type=F32)
         + jnp.dot(agg, w2_ref[...], preferred_element_type=F32)
         + bn_ref[...][0:1, :])
    y = x + _silu(u)
    m = jnp.mean(y, axis=1, keepdims=True)
    yc = y - m
    var = jnp.mean(yc * yc, axis=1, keepdims=True)
    y_ref[...] = (yc * lax.rsqrt(var + 1e-5) * g_ref[...][0:1, :]
                  + b_ref[...][0:1, :])


def _head_a(x_ref, nid_ref, wv_ref, pooled_ref, sy_ref):
    pid = pl.program_id(0)
    O = _onehot(nid_ref[...])
    x = x_ref[...]
    y = jnp.dot(x, wv_ref[...], preferred_element_type=F32)

    @pl.when(pid == 0)
    def _():
        pooled_ref[...] = jnp.zeros((G, H), F32)
        sy_ref[...] = jnp.zeros((G, H), F32)

    pooled_ref[...] += _dotT(O, x)
    sy_ref[...] += _dotT(O, y)


def _head_b(x_ref, nid_ref, tv_ref, wv_ref, cnt_ref, sy_ref, pooled_ref,
            wl_ref, tl_ref, loss_ref):
    pid = pl.program_id(0)
    O = _onehot(nid_ref[...])
    x = x_ref[...]
    y = jnp.dot(x, wv_ref[...], preferred_element_type=F32)
    cntc = jnp.maximum(cnt_ref[...], 1.0)
    meany = sy_ref[...] / cntc
    pv = y - jnp.dot(O, meany, preferred_element_type=F32)
    d = pv - tv_ref[...]
    s = (jnp.sum(d * d) / (3.0 * N)).reshape(1, 1)

    @pl.when(pid == 0)
    def _():
        loss_ref[...] = jnp.zeros((1, 1), F32)

    loss_ref[...] += s

    @pl.when(pid == (N // NB) - 1)
    def _():
        pm = pooled_ref[...] / cntc
        plh = jnp.dot(pm, wl_ref[...], preferred_element_type=F32)
        dl = plh - tl_ref[...]
        loss_ref[...] += (jnp.sum(dl * dl) / (9.0 * G)).reshape(1, 1)


def _dd_body(fpad_hbm, src_hbm, dst_hbm, dd_hbm, idx_s, idx_d, fs, fd, dv,
             sem):
    c = lax.axis_index("c")
    s = lax.axis_index("s")
    wid = s * NC + c

    def blk(bi, carry):
        b = wid + bi * NW

        @pl.when(b < NB_E)
        def _():
            e0 = b * EB
            pltpu.sync_copy(src_hbm.at[pl.ds(e0, EB)], idx_s)
            pltpu.sync_copy(dst_hbm.at[pl.ds(e0, EB)], idx_d)
            pltpu.async_copy(fpad_hbm.at[idx_s], fs, sem).wait()
            pltpu.async_copy(fpad_hbm.at[idx_d], fd, sem).wait()

            def row(r, carry2):
                sl = pl.ds(0, 16)
                d = fs[r, sl] - fd[r, sl]
                d = jnp.where(d > 0.5, d - 1.0,
                              jnp.where(d < -0.5, d + 1.0, d))
                dv[r, sl] = d
                return carry2

            lax.fori_loop(0, EB, row, 0)
            pltpu.sync_copy(dv, dd_hbm.at[pl.ds(e0, EB)])

        return carry

    lax.fori_loop(0, NB_W, blk, 0)


def _edge_body(xs_hbm, xd_hbm, ffw_hbm, src_hbm, dst_hbm, out_hbm, m_hbm,
               idx_s, idx_d, gs, gd, fw, zb, agg_sh, sem):
    c = lax.axis_index("c")
    s = lax.axis_index("s")
    wid = s * NC + c
    zvec = jnp.zeros((16,), F32)

    def zrow(r, carry):
        for k2 in range(8):
            zb[r, pl.ds(k2 * 16, 16)] = zvec
        return carry

    lax.fori_loop(0, ZR, zrow, 0)

    def zero_agg():
        pltpu.sync_copy(zb, agg_sh.at[pl.ds(s * ZPS, ZR)])
        pltpu.sync_copy(zb, agg_sh.at[pl.ds(s * ZPS + ZR, ZR)])
        pltpu.sync_copy(zb.at[pl.ds(0, ZPS - 2 * ZR)],
                        agg_sh.at[pl.ds(s * ZPS + 2 * ZR, ZPS - 2 * ZR)])
        plsc.subcore_barrier()

    def scatter(base):
        def adj(k2, carry3):
            sl = pl.ds(k2 * 16, 16)
            tl = idx_d[sl] - base
            ok = (tl >= 0) & (tl < AGG_HALF)
            idx_d[sl] = jnp.where(ok, tl, AGG_HALF)
            return carry3

        lax.fori_loop(0, 8, adj, 0)
        pltpu.sync_copy(gs, agg_sh.at[idx_d], add=True)

    def dump_agg(base):
        plsc.subcore_barrier()
        pltpu.sync_copy(agg_sh.at[pl.ds(s * CPS, CPS)],
                        out_hbm.at[c, pl.ds(base + s * CPS, CPS)])
        plsc.subcore_barrier()

    zero_agg()

    def blk0(bi, carry):
        b = wid + bi * NW

        @pl.when(b < NB_E)
        def _():
            e0 = b * EB
            cs = pltpu.async_copy(src_hbm.at[pl.ds(e0, EB)], idx_s, sem)
            cd = pltpu.async_copy(dst_hbm.at[pl.ds(e0, EB)], idx_d, sem)
            cs.wait()
            cd.wait()
            c1 = pltpu.async_copy(xs_hbm.at[idx_s], gs, sem)
            c2 = pltpu.async_copy(xd_hbm.at[idx_d], gd, sem)
            c3 = pltpu.async_copy(ffw_hbm.at[pl.ds(e0, EB)], fw, sem)
            c1.wait()
            c2.wait()
            c3.wait()

            @plsc.parallel_loop(0, EB, step=1, unroll=4)
            def row(r):
                for k2 in range(8):
                    sl = pl.ds(k2 * 16, 16)
                    v = gs[r, sl] + gd[r, sl] + fw[r, sl]
                    gs[r, sl] = v / (1.0 + jnp.exp(-v))

            pltpu.sync_copy(gs, m_hbm.at[pl.ds(e0, EB)])
            scatter(0)

        return carry

    lax.fori_loop(0, NB_W, blk0, 0)
    dump_agg(0)

    zero_agg()

    def blk1(bi, carry):
        b = wid + bi * NW

        @pl.when(b < NB_E)
        def _():
            e0 = b * EB
            c1 = pltpu.async_copy(dst_hbm.at[pl.ds(e0, EB)], idx_d, sem)
            c2 = pltpu.async_copy(m_hbm.at[pl.ds(e0, EB)], gs, sem)
            c1.wait()
            c2.wait()
            scatter(AGG_HALF)

        return carry

    lax.fori_loop(0, NB_W, blk1, 0)
    dump_agg(AGG_HALF)


import functools


@functools.lru_cache(maxsize=None)
def _sc_mesh():
    return plsc.VectorSubcoreMesh(core_axis_name="c", subcore_axis_name="s",
                                  num_cores=NC, num_subcores=NS)


@functools.lru_cache(maxsize=None)
def _dd_kernel():
    return pl.kernel(
        _dd_body,
        out_type=jax.ShapeDtypeStruct((E, H), F32),
        mesh=_sc_mesh(),
        scratch_types=[
            pltpu.VMEM((EB,), jnp.int32),
            pltpu.VMEM((EB,), jnp.int32),
            pltpu.VMEM((EB, H), F32),
            pltpu.VMEM((EB, H), F32),
            pltpu.VMEM((EB, H), F32),
            pltpu.SemaphoreType.DMA,
        ],
    )


@functools.lru_cache(maxsize=None)
def _edge_kernel():
    return pl.kernel(
        _edge_body,
        out_type=(jax.ShapeDtypeStruct((NC, NPAD, H), F32),
                  jax.ShapeDtypeStruct((E, H), F32)),
        mesh=_sc_mesh(),
        scratch_types=[
            pltpu.VMEM((EB,), jnp.int32),
            pltpu.VMEM((EB,), jnp.int32),
            pltpu.VMEM((EB, H), F32),
            pltpu.VMEM((EB, H), F32),
            pltpu.VMEM((EB, H), F32),
            pltpu.VMEM((ZR, H), F32),
            pltpu.VMEM_SHARED((AGG_ROWS, H), F32),
            pltpu.SemaphoreType.DMA,
        ],
    )


def _dd_call(fpad, src, dst):
    return _dd_kernel()(fpad, src, dst)


def _edge_call(xs, xd, ffw, src, dst):
    aggp, _ = _edge_kernel()(xs, xd, ffw, src, dst)
    return aggp


def kernel(t, pos, h, l, W_emb, b_emb, W_t, W_lin, W_vin, We_s, We_d, We_f,
           be, Wn, bn, ln_g, ln_b, W_vout, W_lout, node_index,
           edge_node_index):
    nkey = jax.random.key(1)
    eps_l = jax.random.normal(jax.random.fold_in(nkey, 0), l.shape, dtype=F32)
    eps_r = jax.random.normal(jax.random.fold_in(nkey, 1), pos.shape, dtype=F32)
    eps_v = jax.random.normal(jax.random.fold_in(nkey, 2), pos.shape, dtype=F32)

    padc = lambda a: jnp.pad(a, ((0, 0), (0, H - a.shape[1])))
    padr = lambda a: jnp.pad(a, ((0, H - a.shape[0]), (0, 0)))
    pos_p = padc(pos)
    epsr_p = padc(eps_r)
    epsv_p = padc(eps_v)
    h_p = padc(h)
    wemb_p = padr(W_emb)
    wlin_p = padr(W_lin)
    wvin_p = padr(W_vin)
    wvout_p = padc(W_vout)
    wlout_p = padc(W_lout)
    l9p = padc(l.reshape(G, 9))
    epsl9p = padc(eps_l.reshape(G, 9))
    t_mat = jnp.broadcast_to(t[:, None], (G, H))
    nid_b = jnp.broadcast_to(node_index[:, None], (N, H))
    bemb_m = jnp.broadcast_to(b_emb[None, :], (G, H))
    src = edge_node_index[0]
    dst = edge_node_index[1]

    nblk = pl.BlockSpec((NB, H), lambda i: (i, 0))
    gblk = pl.BlockSpec((G, H), lambda i: (0, 0))
    wblk = pl.BlockSpec((H, H), lambda i: (0, 0))

    cnt, sv, su, fpad = pl.pallas_call(
        _prep_a,
        grid=(N // NB,),
        in_specs=[nblk, gblk, nblk, nblk, nblk],
        out_specs=[gblk, gblk, gblk, nblk],
        out_shape=[jax.ShapeDtypeStruct((G, H), F32)] * 3
        + [jax.ShapeDtypeStruct((N, H), F32)],
    )(nid_b, t_mat, pos_p, epsr_p, epsv_p)

    x0, tv = pl.pallas_call(
        _prep_b,
        grid=(N // NB,),
        in_specs=[nblk, gblk, nblk, nblk, nblk, nblk,
                  wblk, wblk, wblk, wblk, gblk, gblk, gblk,
                  gblk, gblk, gblk],
        out_specs=[nblk, nblk],
        out_shape=[jax.ShapeDtypeStruct((N, H), F32)] * 2,
    )(nid_b, t_mat, pos_p, epsr_p, epsv_p, h_p, wemb_p, W_t, wlin_p,
      wvin_p, l9p, epsl9p, bemb_m, cnt, sv, su)

    dd = _dd_call(fpad, src, dst)

    eblk = pl.BlockSpec((BE, H), lambda i: (i, 0))
    ffw = pl.pallas_call(
        _fourier,
        grid=(E // BE,),
        in_specs=[eblk,
                  pl.BlockSpec((NL, 768, H), lambda i: (0, 0, 0))],
        out_specs=[eblk, eblk, eblk, eblk],
        out_shape=[jax.ShapeDtypeStruct((E, H), F32)] * NL,
    )(dd, We_f.astype(jnp.bfloat16))

    n2blk = pl.BlockSpec((NB2, H), lambda i: (i, 0))
    rowblk = pl.BlockSpec((8, H), lambda i: (0, 0))
    x = x0
    for i in range(NL):
        be_b = jnp.broadcast_to(be[i][None, :], (8, H))
        bn_b = jnp.broadcast_to(bn[i][None, :], (8, H))
        g_b = jnp.broadcast_to(ln_g[i][None, :], (8, H))
        b_b = jnp.broadcast_to(ln_b[i][None, :], (8, H))
        xs, xd = pl.pallas_call(
            _node_mm,
            grid=(N // NB2,),
            in_specs=[n2blk, wblk, wblk, rowblk],
            out_specs=[n2blk, n2blk],
            out_shape=[jax.ShapeDtypeStruct((N, H), F32)] * 2,
        )(x, We_s[i], We_d[i], be_b)

        aggp = _edge_call(xs, xd, ffw[i], src, dst)

        x = pl.pallas_call(
            _update,
            grid=(N // NB2,),
            in_specs=[n2blk,
                      pl.BlockSpec((1, NB2, H), lambda i: (0, i, 0)),
                      pl.BlockSpec((1, NB2, H), lambda i: (1, i, 0)),
                      wblk, wblk, rowblk, rowblk, rowblk],
            out_specs=n2blk,
            out_shape=jax.ShapeDtypeStruct((N, H), F32),
        )(x, aggp, aggp, Wn[i, :H, :], Wn[i, H:, :], bn_b, g_b, b_b)

    pooled, sy = pl.pallas_call(
        _head_a,
        grid=(N // NB,),
        in_specs=[nblk, nblk, wblk],
        out_specs=[gblk, gblk],
        out_shape=[jax.ShapeDtypeStruct((G, H), F32)] * 2,
    )(x, nid_b, wvout_p)

    loss = pl.pallas_call(
        _head_b,
        grid=(N // NB,),
        in_specs=[nblk, nblk, nblk, wblk, gblk, gblk, gblk, wblk, gblk],
        out_specs=pl.BlockSpec((1, 1), lambda i: (0, 0)),
        out_shape=jax.ShapeDtypeStruct((1, 1), F32),
    )(x, nid_b, tv, wvout_p, cnt, sy, pooled, wlout_p, epsl9p)

    return loss[0, 0]


# speedup vs baseline: 2.4614x; 1.0368x over previous
import math

import jax
import jax.numpy as jnp
from jax import lax
from jax.experimental import pallas as pl
from jax.experimental.pallas import tpu as pltpu
from jax.experimental.pallas import tpu_sc as plsc

F32 = jnp.float32
G = 256
N = 10000
E = 160000
H = 128
NL = 4
TEPS = 1e-3
B0, B1 = 0.1, 20.0
LOGRATIO = math.log(1.0 / 0.01)
SMIN = 0.01
LOG1E4 = math.log(10000.0)
TWO_PI = 2.0 * math.pi

NB = 1000
NB2 = 2000
BE = 2000

NC = 2
NS = 16
NW = NC * NS
EB = 128
NB_E = E // EB
NB_W = -(-NB_E // NW)
NPAD = 10240
AGG_HALF = NPAD // 2
AGG_ROWS = AGG_HALF + EB
ZPS = AGG_ROWS // NS
CPS = AGG_HALF // NS
ZR = 128


def _silu(v):
    return v / (1.0 + jnp.exp(-v))


def _onehot(nid_blk):
    ni = nid_blk[:, 0:1]
    cols = lax.broadcasted_iota(jnp.int32, (nid_blk.shape[0], G), 1)
    return (ni == cols).astype(F32)


def _dotT(a, b):
    return lax.dot_general(a, b, (((0,), (0,)), ((), ())),
                           preferred_element_type=F32)


def _tgmat(t_mat):
    return TEPS + (1.0 - 2.0 * TEPS) * t_mat


def _node_noise(O, tg_mat, pos_blk, epsr_blk):
    t_node = jnp.dot(O, tg_mat, preferred_element_type=F32)
    sig = SMIN * jnp.exp(t_node * LOGRATIO)
    ft = pos_blk + sig * epsr_blk
    ft = ft - jnp.floor(ft)
    d0 = ft - pos_blk
    rt = jnp.where(d0 > 0.5, d0 - 1.0, jnp.where(d0 < -0.5, d0 + 1.0, d0))
    u = -rt / (sig * sig)
    return ft, u


def _prep_a(nid_ref, t_ref, pos_ref, epsr_ref, epsv_ref,
            cnt_ref, sv_ref, su_ref, fpad_ref):
    pid = pl.program_id(0)
    O = _onehot(nid_ref[...])
    tg_mat = _tgmat(t_ref[...])
    ft, u = _node_noise(O, tg_mat, pos_ref[...], epsr_ref[...])
    fpad_ref[...] = ft
    ones = jnp.ones((NB, H), F32)

    @pl.when(pid == 0)
    def _():
        cnt_ref[...] = jnp.zeros((G, H), F32)
        sv_ref[...] = jnp.zeros((G, H), F32)
        su_ref[...] = jnp.zeros((G, H), F32)

    cnt_ref[...] += _dotT(O, ones)
    sv_ref[...] += _dotT(O, epsv_ref[...])
    su_ref[...] += _dotT(O, u)


def _prep_b(nid_ref, t_ref, pos_ref, epsr_ref, epsv_ref, h_ref,
            wemb_ref, wt_ref, wlin_ref, wvin_ref, l9_ref, epsl9_ref,
            bemb_ref, cnt_ref, sv_ref, su_ref, x0_ref, tv_ref):
    O = _onehot(nid_ref[...])
    tg_mat = _tgmat(t_ref[...])
    _, u = _node_noise(O, tg_mat, pos_ref[...], epsr_ref[...])
    cntc = jnp.maximum(cnt_ref[...], 1.0)
    mean_v = sv_ref[...] / cntc
    mean_u = su_ref[...] / cntc
    v_t = epsv_ref[...] - jnp.dot(O, mean_v, preferred_element_type=F32)
    tv_ref[...] = u - jnp.dot(O, mean_u, preferred_element_type=F32) - v_t
    idx = lax.broadcasted_iota(jnp.int32, (G, H), 1)
    k = jnp.where(idx < 64, idx, idx - 64).astype(F32)
    fr = jnp.exp(-LOG1E4 * k / 63.0)
    a = tg_mat * fr
    temb_in = jnp.where(idx < 64, jnp.sin(a), jnp.cos(a))
    log_ab = -0.25 * tg_mat * tg_mat * (B1 - B0) - 0.5 * tg_mat * B0
    ab = jnp.exp(log_ab)
    ltp = jnp.sqrt(ab) * l9_ref[...] + jnp.sqrt(1.0 - ab) * epsl9_ref[...]
    gnode = (jnp.dot(temb_in, wt_ref[...], preferred_element_type=F32)
             + jnp.dot(ltp, wlin_ref[...], preferred_element_type=F32)
             + bemb_ref[...])
    x0_ref[...] = (jnp.dot(h_ref[...], wemb_ref[...], preferred_element_type=F32)
                   + jnp.dot(O, gnode, preferred_element_type=F32)
                   + jnp.dot(v_t, wvin_ref[...], preferred_element_type=F32))


def _fourier(dd_ref, wf_ref, o0_ref, o1_ref, o2_ref, o3_ref):
    dd = dd_ref[...]
    kv = (lax.broadcasted_iota(jnp.int32, (BE, H), 1) + 1).astype(F32)
    parts = []
    for j in range(3):
        aj = TWO_PI * dd[:, j:j + 1] * kv
        parts.append(jnp.sin(aj))
        parts.append(jnp.cos(aj))
    ff = jnp.concatenate(parts, axis=1).astype(jnp.bfloat16)
    outs = (o0_ref, o1_ref, o2_ref, o3_ref)
    for i in range(NL):
        outs[i][...] = jnp.dot(ff, wf_ref[i], preferred_element_type=F32)


def _node_mm(x_ref, ws_ref, wd_ref, be_ref, xs_ref, xd_ref):
    x = x_ref[...]
    xs_ref[...] = jnp.dot(x, ws_ref[...], preferred_element_type=F32)
    xd_ref[...] = (jnp.dot(x, wd_ref[...], preferred_element_type=F32)
                   + be_ref[...][0:1, :])


def _update(x_ref, a0_ref, a1_ref, w1_ref, w2_ref, bn_ref, g_ref, b_ref,
            y_ref):
    x = x_ref[...]
    agg = a0_ref[0] + a1_ref[0]
    u = (jnp.dot(x, w1_ref[...], preferred_element_type=F32)
         + jnp.dot(agg, w2_ref[...], preferred_element_type=F32)
         + bn_ref[...][0:1, :])
    y = x + _silu(u)
    m = jnp.mean(y, axis=1, keepdims=True)
    yc = y - m
    var = jnp.mean(yc * yc, axis=1, keepdims=True)
    y_ref[...] = (yc * lax.rsqrt(var + 1e-5) * g_ref[...][0:1, :]
                  + b_ref[...][0:1, :])


def _head_a(x_ref, nid_ref, wv_ref, pooled_ref, sy_ref):
    pid = pl.program_id(0)
    O = _onehot(nid_ref[...])
    x = x_ref[...]
    y = jnp.dot(x, wv_ref[...], preferred_element_type=F32)

    @pl.when(pid == 0)
    def _():
        pooled_ref[...] = jnp.zeros((G, H), F32)
        sy_ref[...] = jnp.zeros((G, H), F32)

    pooled_ref[...] += _dotT(O, x)
    sy_ref[...] += _dotT(O, y)


def _head_b(x_ref, nid_ref, tv_ref, wv_ref, cnt_ref, sy_ref, pooled_ref,
            wl_ref, tl_ref, loss_ref):
    pid = pl.program_id(0)
    O = _onehot(nid_ref[...])
    x = x_ref[...]
    y = jnp.dot(x, wv_ref[...], preferred_element_type=F32)
    cntc = jnp.maximum(cnt_ref[...], 1.0)
    meany = sy_ref[...] / cntc
    pv = y - jnp.dot(O, meany, preferred_element_type=F32)
    d = pv - tv_ref[...]
    s = (jnp.sum(d * d) / (3.0 * N)).reshape(1, 1)

    @pl.when(pid == 0)
    def _():
        loss_ref[...] = jnp.zeros((1, 1), F32)

    loss_ref[...] += s

    @pl.when(pid == (N // NB) - 1)
    def _():
        pm = pooled_ref[...] / cntc
        plh = jnp.dot(pm, wl_ref[...], preferred_element_type=F32)
        dl = plh - tl_ref[...]
        loss_ref[...] += (jnp.sum(dl * dl) / (9.0 * G)).reshape(1, 1)


def _dd_body(fpad_hbm, src_hbm, dst_hbm, dd_hbm, idx_s, idx_d, fs, fd, dv,
             sem):
    c = lax.axis_index("c")
    s = lax.axis_index("s")
    wid = s * NC + c

    def blk(bi, carry):
        b = wid + bi * NW

        @pl.when(b < NB_E)
        def _():
            e0 = b * EB
            pltpu.sync_copy(src_hbm.at[pl.ds(e0, EB)], idx_s)
            pltpu.sync_copy(dst_hbm.at[pl.ds(e0, EB)], idx_d)
            pltpu.async_copy(fpad_hbm.at[idx_s], fs, sem).wait()
            pltpu.async_copy(fpad_hbm.at[idx_d], fd, sem).wait()

            def row(r, carry2):
                sl = pl.ds(0, 16)
                d = fs[r, sl] - fd[r, sl]
                d = jnp.where(d > 0.5, d - 1.0,
                              jnp.where(d < -0.5, d + 1.0, d))
                dv[r, sl] = d
                return carry2

            lax.fori_loop(0, EB, row, 0)
            pltpu.sync_copy(dv, dd_hbm.at[pl.ds(e0, EB)])

        return carry

    lax.fori_loop(0, NB_W, blk, 0)


def _edge_body(xs_hbm, xd_hbm, ffw_hbm, src_hbm, dst_hbm, out_hbm, m_hbm,
               idx_s0, idx_d0, gs0, gd0, idx_s1, idx_d1, gs1, gd1, fw,
               agg_sh, sem0, sem1):
    c = lax.axis_index("c")
    s = lax.axis_index("s")
    wid = s * NC + c
    idx_s = (idx_s0, idx_s1)
    idx_d = (idx_d0, idx_d1)
    gs = (gs0, gs1)
    gd = (gd0, gd1)
    sem = (sem0, sem1)
    zvec = jnp.zeros((16,), F32)

    def zero_agg():
        def zrow(r, carry):
            for k2 in range(8):
                gs0[r, pl.ds(k2 * 16, 16)] = zvec
            return carry

        lax.fori_loop(0, ZR, zrow, 0)
        pltpu.sync_copy(gs0, agg_sh.at[pl.ds(s * ZPS, ZR)])
        pltpu.sync_copy(gs0, agg_sh.at[pl.ds(s * ZPS + ZR, ZR)])
        pltpu.sync_copy(gs0.at[pl.ds(0, ZPS - 2 * ZR)],
                        agg_sh.at[pl.ds(s * ZPS + 2 * ZR, ZPS - 2 * ZR)])
        plsc.subcore_barrier()

    def scatter(base, ip, mref):
        def adj(k2, carry3):
            sl = pl.ds(k2 * 16, 16)
            tl = idx_d[ip][sl] - base
            ok = (tl >= 0) & (tl < AGG_HALF)
            idx_d[ip][sl] = jnp.where(ok, tl, AGG_HALF)
            return carry3

        lax.fori_loop(0, 8, adj, 0)
        pltpu.sync_copy(mref, agg_sh.at[idx_d[ip]], add=True)

    def dump_agg(base):
        plsc.subcore_barrier()
        pltpu.sync_copy(agg_sh.at[pl.ds(s * CPS, CPS)],
                        out_hbm.at[c, pl.ds(base + s * CPS, CPS)])
        plsc.subcore_barrier()

    def issue0(bi, ip):
        b = wid + bi * NW

        @pl.when(b < NB_E)
        def _():
            e0 = b * EB
            cs = pltpu.async_copy(src_hbm.at[pl.ds(e0, EB)], idx_s[ip],
                                  sem[ip])
            cd = pltpu.async_copy(dst_hbm.at[pl.ds(e0, EB)], idx_d[ip],
                                  sem[ip])
            cs.wait()
            cd.wait()
            pltpu.async_copy(xs_hbm.at[idx_s[ip]], gs[ip], sem[ip])
            pltpu.async_copy(xd_hbm.at[idx_d[ip]], gd[ip], sem[ip])

    def drain0(ip):
        pltpu.make_async_copy(xs_hbm.at[pl.ds(0, EB)], gs[ip], sem[ip]).wait()
        pltpu.make_async_copy(xs_hbm.at[pl.ds(0, EB)], gd[ip], sem[ip]).wait()

    def work0(bi, ip):
        b = wid + bi * NW

        @pl.when(b < NB_E)
        def _():
            e0 = b * EB
            pltpu.sync_copy(ffw_hbm.at[pl.ds(e0, EB)], fw)
            drain0(ip)

            @plsc.parallel_loop(0, EB, step=1, unroll=4)
            def row(r):
                for k2 in range(8):
                    sl = pl.ds(k2 * 16, 16)
                    v = gs[ip][r, sl] + gd[ip][r, sl] + fw[r, sl]
                    gs[ip][r, sl] = v / (1.0 + jnp.exp(-v))

            pltpu.sync_copy(gs[ip], m_hbm.at[pl.ds(e0, EB)])
            scatter(0, ip, gs[ip])

    zero_agg()
    issue0(0, 0)

    def blk0(bi2, carry):
        for p in range(2):
            bi = bi2 * 2 + p
            issue0(bi + 1, 1 - p)
            work0(bi, p)
        return carry

    lax.fori_loop(0, NB_W // 2, blk0, 0)
    dump_agg(0)

    zero_agg()

    def issue1(bi, ip):
        b = wid + bi * NW

        @pl.when(b < NB_E)
        def _():
            e0 = b * EB
            pltpu.async_copy(dst_hbm.at[pl.ds(e0, EB)], idx_d[ip], sem[ip])
            pltpu.async_copy(m_hbm.at[pl.ds(e0, EB)], gs[ip], sem[ip])

    def work1(bi, ip):
        b = wid + bi * NW

        @pl.when(b < NB_E)
        def _():
            pltpu.make_async_copy(dst_hbm.at[pl.ds(0, EB)], idx_d[ip],
                                  sem[ip]).wait()
            pltpu.make_async_copy(m_hbm.at[pl.ds(0, EB)], gs[ip],
                                  sem[ip]).wait()
            scatter(AGG_HALF, ip, gs[ip])

    issue1(0, 0)

    def blk1(bi2, carry):
        for p in range(2):
            bi = bi2 * 2 + p
            issue1(bi + 1, 1 - p)
            work1(bi, p)
        return carry

    lax.fori_loop(0, NB_W // 2, blk1, 0)
    dump_agg(AGG_HALF)


import functools


@functools.lru_cache(maxsize=None)
def _sc_mesh():
    return plsc.VectorSubcoreMesh(core_axis_name="c", subcore_axis_name="s",
                                  num_cores=NC, num_subcores=NS)


@functools.lru_cache(maxsize=None)
def _dd_kernel():
    return pl.kernel(
        _dd_body,
        out_type=jax.ShapeDtypeStruct((E, H), F32),
        mesh=_sc_mesh(),
        scratch_types=[
            pltpu.VMEM((EB,), jnp.int32),
            pltpu.VMEM((EB,), jnp.int32),
            pltpu.VMEM((EB, H), F32),
            pltpu.VMEM((EB, H), F32),
            pltpu.VMEM((EB, H), F32),
            pltpu.SemaphoreType.DMA,
        ],
    )


@functools.lru_cache(maxsize=None)
def _edge_kernel():
    return pl.kernel(
        _edge_body,
        out_type=(jax.ShapeDtypeStruct((NC, NPAD, H), F32),
                  jax.ShapeDtypeStruct((E, H), F32)),
        mesh=_sc_mesh(),
        scratch_types=[
            pltpu.VMEM((EB,), jnp.int32),
            pltpu.VMEM((EB,), jnp.int32),
            pltpu.VMEM((EB, H), F32),
            pltpu.VMEM((EB, H), F32),
            pltpu.VMEM((EB,), jnp.int32),
            pltpu.VMEM((EB,), jnp.int32),
            pltpu.VMEM((EB, H), F32),
            pltpu.VMEM((EB, H), F32),
            pltpu.VMEM((EB, H), F32),
            pltpu.VMEM_SHARED((AGG_ROWS, H), F32),
            pltpu.SemaphoreType.DMA,
            pltpu.SemaphoreType.DMA,
        ],
    )


def _dd_call(fpad, src, dst):
    return _dd_kernel()(fpad, src, dst)


def _edge_call(xs, xd, ffw, src, dst):
    aggp, _ = _edge_kernel()(xs, xd, ffw, src, dst)
    return aggp


def kernel(t, pos, h, l, W_emb, b_emb, W_t, W_lin, W_vin, We_s, We_d, We_f,
           be, Wn, bn, ln_g, ln_b, W_vout, W_lout, node_index,
           edge_node_index):
    nkey = jax.random.key(1)
    eps_l = jax.random.normal(jax.random.fold_in(nkey, 0), l.shape, dtype=F32)
    eps_r = jax.random.normal(jax.random.fold_in(nkey, 1), pos.shape, dtype=F32)
    eps_v = jax.random.normal(jax.random.fold_in(nkey, 2), pos.shape, dtype=F32)

    padc = lambda a: jnp.pad(a, ((0, 0), (0, H - a.shape[1])))
    padr = lambda a: jnp.pad(a, ((0, H - a.shape[0]), (0, 0)))
    pos_p = padc(pos)
    epsr_p = padc(eps_r)
    epsv_p = padc(eps_v)
    h_p = padc(h)
    wemb_p = padr(W_emb)
    wlin_p = padr(W_lin)
    wvin_p = padr(W_vin)
    wvout_p = padc(W_vout)
    wlout_p = padc(W_lout)
    l9p = padc(l.reshape(G, 9))
    epsl9p = padc(eps_l.reshape(G, 9))
    t_mat = jnp.broadcast_to(t[:, None], (G, H))
    nid_b = jnp.broadcast_to(node_index[:, None], (N, H))
    bemb_m = jnp.broadcast_to(b_emb[None, :], (G, H))
    src = edge_node_index[0]
    dst = edge_node_index[1]

    nblk = pl.BlockSpec((NB, H), lambda i: (i, 0))
    gblk = pl.BlockSpec((G, H), lambda i: (0, 0))
    wblk = pl.BlockSpec((H, H), lambda i: (0, 0))

    cnt, sv, su, fpad = pl.pallas_call(
        _prep_a,
        grid=(N // NB,),
        in_specs=[nblk, gblk, nblk, nblk, nblk],
        out_specs=[gblk, gblk, gblk, nblk],
        out_shape=[jax.ShapeDtypeStruct((G, H), F32)] * 3
        + [jax.ShapeDtypeStruct((N, H), F32)],
    )(nid_b, t_mat, pos_p, epsr_p, epsv_p)

    x0, tv = pl.pallas_call(
        _prep_b,
        grid=(N // NB,),
        in_specs=[nblk, gblk, nblk, nblk, nblk, nblk,
                  wblk, wblk, wblk, wblk, gblk, gblk, gblk,
                  gblk, gblk, gblk],
        out_specs=[nblk, nblk],
        out_shape=[jax.ShapeDtypeStruct((N, H), F32)] * 2,
    )(nid_b, t_mat, pos_p, epsr_p, epsv_p, h_p, wemb_p, W_t, wlin_p,
      wvin_p, l9p, epsl9p, bemb_m, cnt, sv, su)

    dd = _dd_call(fpad, src, dst)

    eblk = pl.BlockSpec((BE, H), lambda i: (i, 0))
    ffw = pl.pallas_call(
        _fourier,
        grid=(E // BE,),
        in_specs=[eblk,
                  pl.BlockSpec((NL, 768, H), lambda i: (0, 0, 0))],
        out_specs=[eblk, eblk, eblk, eblk],
        out_shape=[jax.ShapeDtypeStruct((E, H), F32)] * NL,
    )(dd, We_f.astype(jnp.bfloat16))

    n2blk = pl.BlockSpec((NB2, H), lambda i: (i, 0))
    rowblk = pl.BlockSpec((8, H), lambda i: (0, 0))
    x = x0
    for i in range(NL):
        be_b = jnp.broadcast_to(be[i][None, :], (8, H))
        bn_b = jnp.broadcast_to(bn[i][None, :], (8, H))
        g_b = jnp.broadcast_to(ln_g[i][None, :], (8, H))
        b_b = jnp.broadcast_to(ln_b[i][None, :], (8, H))
        xs, xd = pl.pallas_call(
            _node_mm,
            grid=(N // NB2,),
            in_specs=[n2blk, wblk, wblk, rowblk],
            out_specs=[n2blk, n2blk],
            out_shape=[jax.ShapeDtypeStruct((N, H), F32)] * 2,
        )(x, We_s[i], We_d[i], be_b)

        aggp = _edge_call(xs, xd, ffw[i], src, dst)

        x = pl.pallas_call(
            _update,
            grid=(N // NB2,),
            in_specs=[n2blk,
                      pl.BlockSpec((1, NB2, H), lambda i: (0, i, 0)),
                      pl.BlockSpec((1, NB2, H), lambda i: (1, i, 0)),
                      wblk, wblk, rowblk, rowblk, rowblk],
            out_specs=n2blk,
            out_shape=jax.ShapeDtypeStruct((N, H), F32),
        )(x, aggp, aggp, Wn[i, :H, :], Wn[i, H:, :], bn_b, g_b, b_b)

    pooled, sy = pl.pallas_call(
        _head_a,
        grid=(N // NB,),
        in_specs=[nblk, nblk, wblk],
        out_specs=[gblk, gblk],
        out_shape=[jax.ShapeDtypeStruct((G, H), F32)] * 2,
    )(x, nid_b, wvout_p)

    loss = pl.pallas_call(
        _head_b,
        grid=(N // NB,),
        in_specs=[nblk, nblk, nblk, wblk, gblk, gblk, gblk, wblk, gblk],
        out_specs=pl.BlockSpec((1, 1), lambda i: (0, 0)),
        out_shape=jax.ShapeDtypeStruct((1, 1), F32),
    )(x, nid_b, tv, wvout_p, cnt, sy, pooled, wlout_p, epsl9p)

    return loss[0, 0]


# speedup vs baseline: 2.4758x; 1.0058x over previous
import math

import jax
import jax.numpy as jnp
from jax import lax
from jax.experimental import pallas as pl
from jax.experimental.pallas import tpu as pltpu
from jax.experimental.pallas import tpu_sc as plsc

F32 = jnp.float32
G = 256
N = 10000
E = 160000
H = 128
NL = 4
TEPS = 1e-3
B0, B1 = 0.1, 20.0
LOGRATIO = math.log(1.0 / 0.01)
SMIN = 0.01
LOG1E4 = math.log(10000.0)
TWO_PI = 2.0 * math.pi

NB = 1000
NB2 = 2000
BE = 2000

NC = 2
NS = 16
NW = NC * NS
EB = 128
NB_E = E // EB
NB_W = -(-NB_E // NW)
NPAD = 10240
AGG_HALF = NPAD // 2
AGG_ROWS = AGG_HALF + EB
ZPS = AGG_ROWS // NS
CPS = AGG_HALF // NS
ZR = 128


def _silu(v):
    return v / (1.0 + jnp.exp(-v))


def _onehot(nid_blk):
    ni = nid_blk[:, 0:1]
    cols = lax.broadcasted_iota(jnp.int32, (nid_blk.shape[0], G), 1)
    return (ni == cols).astype(F32)


def _dotT(a, b):
    return lax.dot_general(a, b, (((0,), (0,)), ((), ())),
                           preferred_element_type=F32)


def _tgmat(t_mat):
    return TEPS + (1.0 - 2.0 * TEPS) * t_mat


def _node_noise(O, tg_mat, pos_blk, epsr_blk):
    t_node = jnp.dot(O, tg_mat, preferred_element_type=F32)
    sig = SMIN * jnp.exp(t_node * LOGRATIO)
    ft = pos_blk + sig * epsr_blk
    ft = ft - jnp.floor(ft)
    d0 = ft - pos_blk
    rt = jnp.where(d0 > 0.5, d0 - 1.0, jnp.where(d0 < -0.5, d0 + 1.0, d0))
    u = -rt / (sig * sig)
    return ft, u


def _prep_a(nid_ref, t_ref, pos_ref, epsr_ref, epsv_ref,
            cnt_ref, sv_ref, su_ref, fpad_ref):
    pid = pl.program_id(0)
    O = _onehot(nid_ref[...])
    tg_mat = _tgmat(t_ref[...])
    ft, u = _node_noise(O, tg_mat, pos_ref[...], epsr_ref[...])
    fpad_ref[...] = ft
    ones = jnp.ones((NB, H), F32)

    @pl.when(pid == 0)
    def _():
        cnt_ref[...] = jnp.zeros((G, H), F32)
        sv_ref[...] = jnp.zeros((G, H), F32)
        su_ref[...] = jnp.zeros((G, H), F32)

    cnt_ref[...] += _dotT(O, ones)
    sv_ref[...] += _dotT(O, epsv_ref[...])
    su_ref[...] += _dotT(O, u)


def _prep_b(nid_ref, t_ref, pos_ref, epsr_ref, epsv_ref, h_ref,
            wemb_ref, wt_ref, wlin_ref, wvin_ref, l9_ref, epsl9_ref,
            bemb_ref, cnt_ref, sv_ref, su_ref, x0_ref, tv_ref):
    O = _onehot(nid_ref[...])
    tg_mat = _tgmat(t_ref[...])
    _, u = _node_noise(O, tg_mat, pos_ref[...], epsr_ref[...])
    cntc = jnp.maximum(cnt_ref[...], 1.0)
    mean_v = sv_ref[...] / cntc
    mean_u = su_ref[...] / cntc
    v_t = epsv_ref[...] - jnp.dot(O, mean_v, preferred_element_type=F32)
    tv_ref[...] = u - jnp.dot(O, mean_u, preferred_element_type=F32) - v_t
    idx = lax.broadcasted_iota(jnp.int32, (G, H), 1)
    k = jnp.where(idx < 64, idx, idx - 64).astype(F32)
    fr = jnp.exp(-LOG1E4 * k / 63.0)
    a = tg_mat * fr
    temb_in = jnp.where(idx < 64, jnp.sin(a), jnp.cos(a))
    log_ab = -0.25 * tg_mat * tg_mat * (B1 - B0) - 0.5 * tg_mat * B0
    ab = jnp.exp(log_ab)
    ltp = jnp.sqrt(ab) * l9_ref[...] + jnp.sqrt(1.0 - ab) * epsl9_ref[...]
    gnode = (jnp.dot(temb_in, wt_ref[...], preferred_element_type=F32)
             + jnp.dot(ltp, wlin_ref[...], preferred_element_type=F32)
             + bemb_ref[...])
    x0_ref[...] = (jnp.dot(h_ref[...], wemb_ref[...], preferred_element_type=F32)
                   + jnp.dot(O, gnode, preferred_element_type=F32)
                   + jnp.dot(v_t, wvin_ref[...], preferred_element_type=F32))


def _fourier(dd_ref, wf_ref, o0_ref, o1_ref, o2_ref, o3_ref):
    dd = dd_ref[...]
    kv = (lax.broadcasted_iota(jnp.int32, (BE, H), 1) + 1).astype(F32)
    parts = []
    for j in range(3):
        aj = TWO_PI * dd[:, j:j + 1] * kv
        parts.append(jnp.sin(aj))
        parts.append(jnp.cos(aj))
    ff = jnp.concatenate(parts, axis=1).astype(jnp.bfloat16)
    outs = (o0_ref, o1_ref, o2_ref, o3_ref)
    for i in range(NL):
        outs[i][...] = jnp.dot(ff, wf_ref[i], preferred_element_type=F32)


def _node_mm(x_ref, ws_ref, wd_ref, be_ref, xs_ref, xd_ref):
    x = x_ref[...]
    xs_ref[...] = jnp.dot(x, ws_ref[...], preferred_element_type=F32)
    xd_ref[...] = (jnp.dot(x, wd_ref[...], preferred_element_type=F32)
                   + be_ref[...][0:1, :])


def _update(x_ref, a0_ref, a1_ref, w1_ref, w2_ref, bn_ref, g_ref, b_ref,
            y_ref):
    x = x_ref[...]
    agg = a0_ref[0] + a1_ref[0]
    u = (jnp.dot(x, w1_ref[...], preferred_element_type=F32)
         + jnp.dot(agg, w2_ref[...], preferred_element_type=F32)
         + bn_ref[...][0:1, :])
    y = x + _silu(u)
    m = jnp.mean(y, axis=1, keepdims=True)
    yc = y - m
    var = jnp.mean(yc * yc, axis=1, keepdims=True)
    y_ref[...] = (yc * lax.rsqrt(var + 1e-5) * g_ref[...][0:1, :]
                  + b_ref[...][0:1, :])


def _update_mm(x_ref, a0_ref, a1_ref, w1_ref, w2_ref, bn_ref, g_ref, b_ref,
               ws_ref, wd_ref, be_ref, y_ref, xs_ref, xd_ref):
    x = x_ref[...]
    agg = a0_ref[0] + a1_ref[0]
    u = (jnp.dot(x, w1_ref[...], preferred_element_type=F32)
         + jnp.dot(agg, w2_ref[...], preferred_element_type=F32)
         + bn_ref[...][0:1, :])
    y = x + _silu(u)
    m = jnp.mean(y, axis=1, keepdims=True)
    yc = y - m
    var = jnp.mean(yc * yc, axis=1, keepdims=True)
    y = (yc * lax.rsqrt(var + 1e-5) * g_ref[...][0:1, :]
         + b_ref[...][0:1, :])
    y_ref[...] = y
    xs_ref[...] = jnp.dot(y, ws_ref[...], preferred_element_type=F32)
    xd_ref[...] = (jnp.dot(y, wd_ref[...], preferred_element_type=F32)
                   + be_ref[...][0:1, :])


def _head_a(x_ref, nid_ref, wv_ref, pooled_ref, sy_ref):
    pid = pl.program_id(0)
    O = _onehot(nid_ref[...])
    x = x_ref[...]
    y = jnp.dot(x, wv_ref[...], preferred_element_type=F32)

    @pl.when(pid == 0)
    def _():
        pooled_ref[...] = jnp.zeros((G, H), F32)
        sy_ref[...] = jnp.zeros((G, H), F32)

    pooled_ref[...] += _dotT(O, x)
    sy_ref[...] += _dotT(O, y)


def _head_b(x_ref, nid_ref, tv_ref, wv_ref, cnt_ref, sy_ref, pooled_ref,
            wl_ref, tl_ref, loss_ref):
    pid = pl.program_id(0)
    O = _onehot(nid_ref[...])
    x = x_ref[...]
    y = jnp.dot(x, wv_ref[...], preferred_element_type=F32)
    cntc = jnp.maximum(cnt_ref[...], 1.0)
    meany = sy_ref[...] / cntc
    pv = y - jnp.dot(O, meany, preferred_element_type=F32)
    d = pv - tv_ref[...]
    s = (jnp.sum(d * d) / (3.0 * N)).reshape(1, 1)

    @pl.when(pid == 0)
    def _():
        loss_ref[...] = jnp.zeros((1, 1), F32)

    loss_ref[...] += s

    @pl.when(pid == (N // NB) - 1)
    def _():
        pm = pooled_ref[...] / cntc
        plh = jnp.dot(pm, wl_ref[...], preferred_element_type=F32)
        dl = plh - tl_ref[...]
        loss_ref[...] += (jnp.sum(dl * dl) / (9.0 * G)).reshape(1, 1)


def _dd_body(fpad_hbm, src_hbm, dst_hbm, dd_hbm, idx_s, idx_d, fs, fd, dv,
             sem):
    c = lax.axis_index("c")
    s = lax.axis_index("s")
    wid = s * NC + c

    def blk(bi, carry):
        b = wid + bi * NW

        @pl.when(b < NB_E)
        def _():
            e0 = b * EB
            pltpu.sync_copy(src_hbm.at[pl.ds(e0, EB)], idx_s)
            pltpu.sync_copy(dst_hbm.at[pl.ds(e0, EB)], idx_d)
            pltpu.async_copy(fpad_hbm.at[idx_s], fs, sem).wait()
            pltpu.async_copy(fpad_hbm.at[idx_d], fd, sem).wait()

            def row(r, carry2):
                sl = pl.ds(0, 16)
                d = fs[r, sl] - fd[r, sl]
                d = jnp.where(d > 0.5, d - 1.0,
                              jnp.where(d < -0.5, d + 1.0, d))
                dv[r, sl] = d
                return carry2

            lax.fori_loop(0, EB, row, 0)
            pltpu.sync_copy(dv, dd_hbm.at[pl.ds(e0, EB)])

        return carry

    lax.fori_loop(0, NB_W, blk, 0)


def _edge_body(xs_hbm, xd_hbm, ffw_hbm, src_hbm, dst_hbm, out_hbm, m_hbm,
               idx_s0, idx_d0, gs0, gd0, idx_s1, idx_d1, gs1, gd1, fw,
               agg_sh, sem0, sem1):
    c = lax.axis_index("c")
    s = lax.axis_index("s")
    wid = s * NC + c
    idx_s = (idx_s0, idx_s1)
    idx_d = (idx_d0, idx_d1)
    gs = (gs0, gs1)
    gd = (gd0, gd1)
    sem = (sem0, sem1)
    zvec = jnp.zeros((16,), F32)

    def zero_agg():
        def zrow(r, carry):
            for k2 in range(8):
                gs0[r, pl.ds(k2 * 16, 16)] = zvec
            return carry

        lax.fori_loop(0, ZR, zrow, 0)
        pltpu.sync_copy(gs0, agg_sh.at[pl.ds(s * ZPS, ZR)])
        pltpu.sync_copy(gs0, agg_sh.at[pl.ds(s * ZPS + ZR, ZR)])
        pltpu.sync_copy(gs0.at[pl.ds(0, ZPS - 2 * ZR)],
                        agg_sh.at[pl.ds(s * ZPS + 2 * ZR, ZPS - 2 * ZR)])
        plsc.subcore_barrier()

    def scatter(base, ip, mref):
        def adj(k2, carry3):
            sl = pl.ds(k2 * 16, 16)
            tl = idx_d[ip][sl] - base
            ok = (tl >= 0) & (tl < AGG_HALF)
            idx_d[ip][sl] = jnp.where(ok, tl, AGG_HALF)
            return carry3

        lax.fori_loop(0, 8, adj, 0)
        pltpu.sync_copy(mref, agg_sh.at[idx_d[ip]], add=True)

    def dump_agg(base):
        plsc.subcore_barrier()
        pltpu.sync_copy(agg_sh.at[pl.ds(s * CPS, CPS)],
                        out_hbm.at[c, pl.ds(base + s * CPS, CPS)])
        plsc.subcore_barrier()

    def issue0(bi, ip):
        b = wid + bi * NW

        @pl.when(b < NB_E)
        def _():
            e0 = b * EB
            cs = pltpu.async_copy(src_hbm.at[pl.ds(e0, EB)], idx_s[ip],
                                  sem[ip])
            cd = pltpu.async_copy(dst_hbm.at[pl.ds(e0, EB)], idx_d[ip],
                                  sem[ip])
            cs.wait()
            cd.wait()
            pltpu.async_copy(xs_hbm.at[idx_s[ip]], gs[ip], sem[ip])
            pltpu.async_copy(xd_hbm.at[idx_d[ip]], gd[ip], sem[ip])

    def drain0(ip):
        pltpu.make_async_copy(xs_hbm.at[pl.ds(0, EB)], gs[ip], sem[ip]).wait()
        pltpu.make_async_copy(xs_hbm.at[pl.ds(0, EB)], gd[ip], sem[ip]).wait()

    def work0(bi, ip):
        b = wid + bi * NW

        @pl.when(b < NB_E)
        def _():
            e0 = b * EB
            pltpu.sync_copy(ffw_hbm.at[pl.ds(e0, EB)], fw)
            drain0(ip)

            @plsc.parallel_loop(0, EB, step=1, unroll=4)
            def row(r):
                for k2 in range(8):
                    sl = pl.ds(k2 * 16, 16)
                    v = gs[ip][r, sl] + gd[ip][r, sl] + fw[r, sl]
                    gs[ip][r, sl] = v / (1.0 + jnp.exp(-v))

            pltpu.sync_copy(gs[ip], m_hbm.at[pl.ds(e0, EB)])
            scatter(0, ip, gs[ip])

    zero_agg()
    issue0(0, 0)

    def blk0(bi2, carry):
        for p in range(2):
            bi = bi2 * 2 + p
            issue0(bi + 1, 1 - p)
            work0(bi, p)
        return carry

    lax.fori_loop(0, NB_W // 2, blk0, 0)
    dump_agg(0)

    zero_agg()

    def issue1(bi, ip):
        b = wid + bi * NW

        @pl.when(b < NB_E)
        def _():
            e0 = b * EB
            pltpu.async_copy(dst_hbm.at[pl.ds(e0, EB)], idx_d[ip], sem[ip])
            pltpu.async_copy(m_hbm.at[pl.ds(e0, EB)], gs[ip], sem[ip])

    def work1(bi, ip):
        b = wid + bi * NW

        @pl.when(b < NB_E)
        def _():
            pltpu.make_async_copy(dst_hbm.at[pl.ds(0, EB)], idx_d[ip],
                                  sem[ip]).wait()
            pltpu.make_async_copy(m_hbm.at[pl.ds(0, EB)], gs[ip],
                                  sem[ip]).wait()
            scatter(AGG_HALF, ip, gs[ip])

    issue1(0, 0)

    def blk1(bi2, carry):
        for p in range(2):
            bi = bi2 * 2 + p
            issue1(bi + 1, 1 - p)
            work1(bi, p)
        return carry

    lax.fori_loop(0, NB_W // 2, blk1, 0)
    dump_agg(AGG_HALF)


import functools


@functools.lru_cache(maxsize=None)
def _sc_mesh():
    return plsc.VectorSubcoreMesh(core_axis_name="c", subcore_axis_name="s",
                                  num_cores=NC, num_subcores=NS)


@functools.lru_cache(maxsize=None)
def _dd_kernel():
    return pl.kernel(
        _dd_body,
        out_type=jax.ShapeDtypeStruct((E, H), F32),
        mesh=_sc_mesh(),
        scratch_types=[
            pltpu.VMEM((EB,), jnp.int32),
            pltpu.VMEM((EB,), jnp.int32),
            pltpu.VMEM((EB, H), F32),
            pltpu.VMEM((EB, H), F32),
            pltpu.VMEM((EB, H), F32),
            pltpu.SemaphoreType.DMA,
        ],
    )


@functools.lru_cache(maxsize=None)
def _edge_kernel():
    return pl.kernel(
        _edge_body,
        out_type=(jax.ShapeDtypeStruct((NC, NPAD, H), F32),
                  jax.ShapeDtypeStruct((E, H), F32)),
        mesh=_sc_mesh(),
        scratch_types=[
            pltpu.VMEM((EB,), jnp.int32),
            pltpu.VMEM((EB,), jnp.int32),
            pltpu.VMEM((EB, H), F32),
            pltpu.VMEM((EB, H), F32),
            pltpu.VMEM((EB,), jnp.int32),
            pltpu.VMEM((EB,), jnp.int32),
            pltpu.VMEM((EB, H), F32),
            pltpu.VMEM((EB, H), F32),
            pltpu.VMEM((EB, H), F32),
            pltpu.VMEM_SHARED((AGG_ROWS, H), F32),
            pltpu.SemaphoreType.DMA,
            pltpu.SemaphoreType.DMA,
        ],
    )


def _dd_call(fpad, src, dst):
    return _dd_kernel()(fpad, src, dst)


def _edge_call(xs, xd, ffw, src, dst):
    aggp, _ = _edge_kernel()(xs, xd, ffw, src, dst)
    return aggp


def kernel(t, pos, h, l, W_emb, b_emb, W_t, W_lin, W_vin, We_s, We_d, We_f,
           be, Wn, bn, ln_g, ln_b, W_vout, W_lout, node_index,
           edge_node_index):
    nkey = jax.random.key(1)
    eps_l = jax.random.normal(jax.random.fold_in(nkey, 0), l.shape, dtype=F32)
    eps_r = jax.random.normal(jax.random.fold_in(nkey, 1), pos.shape, dtype=F32)
    eps_v = jax.random.normal(jax.random.fold_in(nkey, 2), pos.shape, dtype=F32)

    padc = lambda a: jnp.pad(a, ((0, 0), (0, H - a.shape[1])))
    padr = lambda a: jnp.pad(a, ((0, H - a.shape[0]), (0, 0)))
    pos_p = padc(pos)
    epsr_p = padc(eps_r)
    epsv_p = padc(eps_v)
    h_p = padc(h)
    wemb_p = padr(W_emb)
    wlin_p = padr(W_lin)
    wvin_p = padr(W_vin)
    wvout_p = padc(W_vout)
    wlout_p = padc(W_lout)
    l9p = padc(l.reshape(G, 9))
    epsl9p = padc(eps_l.reshape(G, 9))
    t_mat = jnp.broadcast_to(t[:, None], (G, H))
    nid_b = jnp.broadcast_to(node_index[:, None], (N, H))
    bemb_m = jnp.broadcast_to(b_emb[None, :], (G, H))
    src = edge_node_index[0]
    dst = edge_node_index[1]

    nblk = pl.BlockSpec((NB, H), lambda i: (i, 0))
    gblk = pl.BlockSpec((G, H), lambda i: (0, 0))
    wblk = pl.BlockSpec((H, H), lambda i: (0, 0))

    cnt, sv, su, fpad = pl.pallas_call(
        _prep_a,
        grid=(N // NB,),
        in_specs=[nblk, gblk, nblk, nblk, nblk],
        out_specs=[gblk, gblk, gblk, nblk],
        out_shape=[jax.ShapeDtypeStruct((G, H), F32)] * 3
        + [jax.ShapeDtypeStruct((N, H), F32)],
    )(nid_b, t_mat, pos_p, epsr_p, epsv_p)

    x0, tv = pl.pallas_call(
        _prep_b,
        grid=(N // NB,),
        in_specs=[nblk, gblk, nblk, nblk, nblk, nblk,
                  wblk, wblk, wblk, wblk, gblk, gblk, gblk,
                  gblk, gblk, gblk],
        out_specs=[nblk, nblk],
        out_shape=[jax.ShapeDtypeStruct((N, H), F32)] * 2,
    )(nid_b, t_mat, pos_p, epsr_p, epsv_p, h_p, wemb_p, W_t, wlin_p,
      wvin_p, l9p, epsl9p, bemb_m, cnt, sv, su)

    dd = _dd_call(fpad, src, dst)

    eblk = pl.BlockSpec((BE, H), lambda i: (i, 0))
    ffw = pl.pallas_call(
        _fourier,
        grid=(E // BE,),
        in_specs=[eblk,
                  pl.BlockSpec((NL, 768, H), lambda i: (0, 0, 0))],
        out_specs=[eblk, eblk, eblk, eblk],
        out_shape=[jax.ShapeDtypeStruct((E, H), F32)] * NL,
    )(dd, We_f.astype(jnp.bfloat16))

    n2blk = pl.BlockSpec((NB2, H), lambda i: (i, 0))
    rowblk = pl.BlockSpec((8, H), lambda i: (0, 0))
    a0blk = pl.BlockSpec((1, NB2, H), lambda i: (0, i, 0))
    a1blk = pl.BlockSpec((1, NB2, H), lambda i: (1, i, 0))
    be_b = [jnp.broadcast_to(be[i][None, :], (8, H)) for i in range(NL)]
    bn_b = [jnp.broadcast_to(bn[i][None, :], (8, H)) for i in range(NL)]
    g_b = [jnp.broadcast_to(ln_g[i][None, :], (8, H)) for i in range(NL)]
    b_b = [jnp.broadcast_to(ln_b[i][None, :], (8, H)) for i in range(NL)]

    x = x0
    xs, xd = pl.pallas_call(
        _node_mm,
        grid=(N // NB2,),
        in_specs=[n2blk, wblk, wblk, rowblk],
        out_specs=[n2blk, n2blk],
        out_shape=[jax.ShapeDtypeStruct((N, H), F32)] * 2,
    )(x, We_s[0], We_d[0], be_b[0])

    for i in range(NL):
        aggp = _edge_call(xs, xd, ffw[i], src, dst)
        if i < NL - 1:
            x, xs, xd = pl.pallas_call(
                _update_mm,
                grid=(N // NB2,),
                in_specs=[n2blk, a0blk, a1blk, wblk, wblk, rowblk, rowblk,
                          rowblk, wblk, wblk, rowblk],
                out_specs=[n2blk, n2blk, n2blk],
                out_shape=[jax.ShapeDtypeStruct((N, H), F32)] * 3,
            )(x, aggp, aggp, Wn[i, :H, :], Wn[i, H:, :], bn_b[i], g_b[i],
              b_b[i], We_s[i + 1], We_d[i + 1], be_b[i + 1])
        else:
            x = pl.pallas_call(
                _update,
                grid=(N // NB2,),
                in_specs=[n2blk, a0blk, a1blk, wblk, wblk, rowblk, rowblk,
                          rowblk],
                out_specs=n2blk,
                out_shape=jax.ShapeDtypeStruct((N, H), F32),
            )(x, aggp, aggp, Wn[i, :H, :], Wn[i, H:, :], bn_b[i], g_b[i],
              b_b[i])

    pooled, sy = pl.pallas_call(
        _head_a,
        grid=(N // NB,),
        in_specs=[nblk, nblk, wblk],
        out_specs=[gblk, gblk],
        out_shape=[jax.ShapeDtypeStruct((G, H), F32)] * 2,
    )(x, nid_b, wvout_p)

    loss = pl.pallas_call(
        _head_b,
        grid=(N // NB,),
        in_specs=[nblk, nblk, nblk, wblk, gblk, gblk, gblk, wblk, gblk],
        out_specs=pl.BlockSpec((1, 1), lambda i: (0, 0)),
        out_shape=jax.ShapeDtypeStruct((1, 1), F32),
    )(x, nid_b, tv, wvout_p, cnt, sy, pooled, wlout_p, epsl9p)

    return loss[0, 0]


# speedup vs baseline: 2.5635x; 1.0354x over previous
import math

import jax
import jax.numpy as jnp
from jax import lax
from jax.experimental import pallas as pl
from jax.experimental.pallas import tpu as pltpu
from jax.experimental.pallas import tpu_sc as plsc

F32 = jnp.float32
G = 256
N = 10000
E = 160000
H = 128
NL = 4
TEPS = 1e-3
B0, B1 = 0.1, 20.0
LOGRATIO = math.log(1.0 / 0.01)
SMIN = 0.01
LOG1E4 = math.log(10000.0)
TWO_PI = 2.0 * math.pi

NB = 1000
NB2 = 2000
BE = 2000

NC = 2
NS = 16
NW = NC * NS
EB = 128
NB_E = E // EB
NB_W = -(-NB_E // NW)
NPAD = 10240
AGG_HALF = NPAD // 2
AGG_ROWS = AGG_HALF + EB
ZPS = AGG_ROWS // NS
CPS = AGG_HALF // NS
ZR = 128


def _silu(v):
    return v / (1.0 + jnp.exp(-v))


def _onehot(nid_blk):
    ni = nid_blk[:, 0:1]
    cols = lax.broadcasted_iota(jnp.int32, (nid_blk.shape[0], G), 1)
    return (ni == cols).astype(F32)


def _dotT(a, b):
    return lax.dot_general(a, b, (((0,), (0,)), ((), ())),
                           preferred_element_type=F32)


def _tgmat(t_mat):
    return TEPS + (1.0 - 2.0 * TEPS) * t_mat


def _node_noise(O, tg_mat, pos_blk, epsr_blk):
    t_node = jnp.dot(O, tg_mat, preferred_element_type=F32)
    sig = SMIN * jnp.exp(t_node * LOGRATIO)
    ft = pos_blk + sig * epsr_blk
    ft = ft - jnp.floor(ft)
    d0 = ft - pos_blk
    rt = jnp.where(d0 > 0.5, d0 - 1.0, jnp.where(d0 < -0.5, d0 + 1.0, d0))
    u = -rt / (sig * sig)
    return ft, u


def _prep_a(nid_ref, t_ref, pos_ref, epsr_ref, epsv_ref,
            cnt_ref, sv_ref, su_ref, fpad_ref):
    pid = pl.program_id(0)
    O = _onehot(nid_ref[...])
    tg_mat = _tgmat(t_ref[...])
    ft, u = _node_noise(O, tg_mat, pos_ref[...], epsr_ref[...])
    fpad_ref[...] = ft
    ones = jnp.ones((NB, H), F32)

    @pl.when(pid == 0)
    def _():
        cnt_ref[...] = jnp.zeros((G, H), F32)
        sv_ref[...] = jnp.zeros((G, H), F32)
        su_ref[...] = jnp.zeros((G, H), F32)

    cnt_ref[...] += _dotT(O, ones)
    sv_ref[...] += _dotT(O, epsv_ref[...])
    su_ref[...] += _dotT(O, u)


def _prep_b(nid_ref, t_ref, pos_ref, epsr_ref, epsv_ref, h_ref,
            wemb_ref, wt_ref, wlin_ref, wvin_ref, l9_ref, epsl9_ref,
            bemb_ref, cnt_ref, sv_ref, su_ref, x0_ref, tv_ref):
    O = _onehot(nid_ref[...])
    tg_mat = _tgmat(t_ref[...])
    _, u = _node_noise(O, tg_mat, pos_ref[...], epsr_ref[...])
    cntc = jnp.maximum(cnt_ref[...], 1.0)
    mean_v = sv_ref[...] / cntc
    mean_u = su_ref[...] / cntc
    v_t = epsv_ref[...] - jnp.dot(O, mean_v, preferred_element_type=F32)
    tv_ref[...] = u - jnp.dot(O, mean_u, preferred_element_type=F32) - v_t
    idx = lax.broadcasted_iota(jnp.int32, (G, H), 1)
    k = jnp.where(idx < 64, idx, idx - 64).astype(F32)
    fr = jnp.exp(-LOG1E4 * k / 63.0)
    a = tg_mat * fr
    temb_in = jnp.where(idx < 64, jnp.sin(a), jnp.cos(a))
    log_ab = -0.25 * tg_mat * tg_mat * (B1 - B0) - 0.5 * tg_mat * B0
    ab = jnp.exp(log_ab)
    ltp = jnp.sqrt(ab) * l9_ref[...] + jnp.sqrt(1.0 - ab) * epsl9_ref[...]
    gnode = (jnp.dot(temb_in, wt_ref[...], preferred_element_type=F32)
             + jnp.dot(ltp, wlin_ref[...], preferred_element_type=F32)
             + bemb_ref[...])
    x0_ref[...] = (jnp.dot(h_ref[...], wemb_ref[...], preferred_element_type=F32)
                   + jnp.dot(O, gnode, preferred_element_type=F32)
                   + jnp.dot(v_t, wvin_ref[...], preferred_element_type=F32))


def _fourier(dd_ref, wf_ref, o0_ref, o1_ref, o2_ref, o3_ref):
    dd = dd_ref[...]
    kv = (lax.broadcasted_iota(jnp.int32, (BE, H), 1) + 1).astype(F32)
    parts = []
    for j in range(3):
        aj = TWO_PI * dd[:, j:j + 1] * kv
        parts.append(jnp.sin(aj))
        parts.append(jnp.cos(aj))
    ff = jnp.concatenate(parts, axis=1).astype(jnp.bfloat16)
    outs = (o0_ref, o1_ref, o2_ref, o3_ref)
    for i in range(NL):
        outs[i][...] = jnp.dot(ff, wf_ref[i], preferred_element_type=F32)


def _node_mm(x_ref, ws_ref, wd_ref, be_ref, xs_ref, xd_ref):
    x = x_ref[...]
    xs_ref[...] = jnp.dot(x, ws_ref[...], preferred_element_type=F32)
    xd_ref[...] = (jnp.dot(x, wd_ref[...], preferred_element_type=F32)
                   + be_ref[...][0:1, :])


def _update(x_ref, a0_ref, a1_ref, w1_ref, w2_ref, bn_ref, g_ref, b_ref,
            y_ref):
    x = x_ref[...]
    agg = a0_ref[0] + a1_ref[0]
    u = (jnp.dot(x, w1_ref[...], preferred_element_type=F32)
         + jnp.dot(agg, w2_ref[...], preferred_element_type=F32)
         + bn_ref[...][0:1, :])
    y = x + _silu(u)
    m = jnp.mean(y, axis=1, keepdims=True)
    yc = y - m
    var = jnp.mean(yc * yc, axis=1, keepdims=True)
    y_ref[...] = (yc * lax.rsqrt(var + 1e-5) * g_ref[...][0:1, :]
                  + b_ref[...][0:1, :])


def _update_mm(x_ref, a0_ref, a1_ref, w1_ref, w2_ref, bn_ref, g_ref, b_ref,
               ws_ref, wd_ref, be_ref, y_ref, xs_ref, xd_ref):
    x = x_ref[...]
    agg = a0_ref[0] + a1_ref[0]
    u = (jnp.dot(x, w1_ref[...], preferred_element_type=F32)
         + jnp.dot(agg, w2_ref[...], preferred_element_type=F32)
         + bn_ref[...][0:1, :])
    y = x + _silu(u)
    m = jnp.mean(y, axis=1, keepdims=True)
    yc = y - m
    var = jnp.mean(yc * yc, axis=1, keepdims=True)
    y = (yc * lax.rsqrt(var + 1e-5) * g_ref[...][0:1, :]
         + b_ref[...][0:1, :])
    y_ref[...] = y
    xs_ref[...] = jnp.dot(y, ws_ref[...], preferred_element_type=F32)
    xd_ref[...] = (jnp.dot(y, wd_ref[...], preferred_element_type=F32)
                   + be_ref[...][0:1, :])


def _head_a(x_ref, nid_ref, wv_ref, pooled_ref, sy_ref):
    pid = pl.program_id(0)
    O = _onehot(nid_ref[...])
    x = x_ref[...]
    y = jnp.dot(x, wv_ref[...], preferred_element_type=F32)

    @pl.when(pid == 0)
    def _():
        pooled_ref[...] = jnp.zeros((G, H), F32)
        sy_ref[...] = jnp.zeros((G, H), F32)

    pooled_ref[...] += _dotT(O, x)
    sy_ref[...] += _dotT(O, y)


def _head_b(x_ref, nid_ref, tv_ref, wv_ref, cnt_ref, sy_ref, pooled_ref,
            wl_ref, tl_ref, loss_ref):
    pid = pl.program_id(0)
    O = _onehot(nid_ref[...])
    x = x_ref[...]
    y = jnp.dot(x, wv_ref[...], preferred_element_type=F32)
    cntc = jnp.maximum(cnt_ref[...], 1.0)
    meany = sy_ref[...] / cntc
    pv = y - jnp.dot(O, meany, preferred_element_type=F32)
    d = pv - tv_ref[...]
    s = (jnp.sum(d * d) / (3.0 * N)).reshape(1, 1)

    @pl.when(pid == 0)
    def _():
        loss_ref[...] = jnp.zeros((1, 1), F32)

    loss_ref[...] += s

    @pl.when(pid == (N // NB) - 1)
    def _():
        pm = pooled_ref[...] / cntc
        plh = jnp.dot(pm, wl_ref[...], preferred_element_type=F32)
        dl = plh - tl_ref[...]
        loss_ref[...] += (jnp.sum(dl * dl) / (9.0 * G)).reshape(1, 1)


def _dd_body(fpad_hbm, src_hbm, dst_hbm, dd_hbm, idx_s0, idx_d0, fs0, fd0,
             idx_s1, idx_d1, fs1, fd1, dv, sem0, sem1):
    c = lax.axis_index("c")
    s = lax.axis_index("s")
    wid = s * NC + c
    idx_s = (idx_s0, idx_s1)
    idx_d = (idx_d0, idx_d1)
    fs = (fs0, fs1)
    fd = (fd0, fd1)
    sem = (sem0, sem1)

    def issue(bi, ip):
        b = wid + bi * NW

        @pl.when(b < NB_E)
        def _():
            e0 = b * EB
            cs = pltpu.async_copy(src_hbm.at[pl.ds(e0, EB)], idx_s[ip],
                                  sem[ip])
            cd = pltpu.async_copy(dst_hbm.at[pl.ds(e0, EB)], idx_d[ip],
                                  sem[ip])
            cs.wait()
            cd.wait()
            pltpu.async_copy(fpad_hbm.at[idx_s[ip]], fs[ip], sem[ip])
            pltpu.async_copy(fpad_hbm.at[idx_d[ip]], fd[ip], sem[ip])

    def work(bi, ip):
        b = wid + bi * NW

        @pl.when(b < NB_E)
        def _():
            e0 = b * EB
            pltpu.make_async_copy(fpad_hbm.at[pl.ds(0, EB)], fs[ip],
                                  sem[ip]).wait()
            pltpu.make_async_copy(fpad_hbm.at[pl.ds(0, EB)], fd[ip],
                                  sem[ip]).wait()

            @plsc.parallel_loop(0, EB, step=1, unroll=4)
            def row(r):
                sl = pl.ds(0, 16)
                d = fs[ip][r, sl] - fd[ip][r, sl]
                d = jnp.where(d > 0.5, d - 1.0,
                              jnp.where(d < -0.5, d + 1.0, d))
                dv[r, sl] = d

            pltpu.sync_copy(dv, dd_hbm.at[pl.ds(e0, EB)])

        return

    issue(0, 0)

    def blk(bi2, carry):
        for p in range(2):
            bi = bi2 * 2 + p
            issue(bi + 1, 1 - p)
            work(bi, p)
        return carry

    lax.fori_loop(0, NB_W // 2, blk, 0)


def _edge_body(xs_hbm, xd_hbm, ffw_hbm, src_hbm, dst_hbm, out_hbm, m_hbm,
               idx_s0, idx_d0, gs0, gd0, idx_s1, idx_d1, gs1, gd1, fw,
               agg_sh, sem0, sem1):
    c = lax.axis_index("c")
    s = lax.axis_index("s")
    wid = s * NC + c
    idx_s = (idx_s0, idx_s1)
    idx_d = (idx_d0, idx_d1)
    gs = (gs0, gs1)
    gd = (gd0, gd1)
    sem = (sem0, sem1)
    zvec = jnp.zeros((16,), F32)

    def zero_agg():
        def zrow(r, carry):
            for k2 in range(8):
                gs0[r, pl.ds(k2 * 16, 16)] = zvec
            return carry

        lax.fori_loop(0, ZR, zrow, 0)
        pltpu.sync_copy(gs0, agg_sh.at[pl.ds(s * ZPS, ZR)])
        pltpu.sync_copy(gs0, agg_sh.at[pl.ds(s * ZPS + ZR, ZR)])
        pltpu.sync_copy(gs0.at[pl.ds(0, ZPS - 2 * ZR)],
                        agg_sh.at[pl.ds(s * ZPS + 2 * ZR, ZPS - 2 * ZR)])
        plsc.subcore_barrier()

    def scatter(base, ip, mref):
        def adj(k2, carry3):
            sl = pl.ds(k2 * 16, 16)
            tl = idx_d[ip][sl] - base
            ok = (tl >= 0) & (tl < AGG_HALF)
            idx_d[ip][sl] = jnp.where(ok, tl, AGG_HALF)
            return carry3

        lax.fori_loop(0, 8, adj, 0)
        pltpu.sync_copy(mref, agg_sh.at[idx_d[ip]], add=True)

    def dump_agg(base):
        plsc.subcore_barrier()
        pltpu.sync_copy(agg_sh.at[pl.ds(s * CPS, CPS)],
                        out_hbm.at[c, pl.ds(base + s * CPS, CPS)])
        plsc.subcore_barrier()

    def issue0(bi, ip):
        b = wid + bi * NW

        @pl.when(b < NB_E)
        def _():
            e0 = b * EB
            cs = pltpu.async_copy(src_hbm.at[pl.ds(e0, EB)], idx_s[ip],
                                  sem[ip])
            cd = pltpu.async_copy(dst_hbm.at[pl.ds(e0, EB)], idx_d[ip],
                                  sem[ip])
            cs.wait()
            cd.wait()
            pltpu.async_copy(xs_hbm.at[idx_s[ip]], gs[ip], sem[ip])
            pltpu.async_copy(xd_hbm.at[idx_d[ip]], gd[ip], sem[ip])

    def drain0(ip):
        pltpu.make_async_copy(xs_hbm.at[pl.ds(0, EB)], gs[ip], sem[ip]).wait()
        pltpu.make_async_copy(xs_hbm.at[pl.ds(0, EB)], gd[ip], sem[ip]).wait()

    def work0(bi, ip):
        b = wid + bi * NW

        @pl.when(b < NB_E)
        def _():
            e0 = b * EB
            pltpu.sync_copy(ffw_hbm.at[pl.ds(e0, EB)], fw)
            drain0(ip)

            @plsc.parallel_loop(0, EB, step=1, unroll=4)
            def row(r):
                for k2 in range(8):
                    sl = pl.ds(k2 * 16, 16)
                    v = gs[ip][r, sl] + gd[ip][r, sl] + fw[r, sl]
                    gs[ip][r, sl] = v / (1.0 + jnp.exp(-v))

            pltpu.sync_copy(gs[ip], m_hbm.at[pl.ds(e0, EB)])
            scatter(0, ip, gs[ip])

    zero_agg()
    issue0(0, 0)

    def blk0(bi2, carry):
        for p in range(2):
            bi = bi2 * 2 + p
            issue0(bi + 1, 1 - p)
            work0(bi, p)
        return carry

    lax.fori_loop(0, NB_W // 2, blk0, 0)
    dump_agg(0)

    zero_agg()

    def issue1(bi, ip):
        b = wid + bi * NW

        @pl.when(b < NB_E)
        def _():
            e0 = b * EB
            pltpu.async_copy(dst_hbm.at[pl.ds(e0, EB)], idx_d[ip], sem[ip])
            pltpu.async_copy(m_hbm.at[pl.ds(e0, EB)], gs[ip], sem[ip])

    def work1(bi, ip):
        b = wid + bi * NW

        @pl.when(b < NB_E)
        def _():
            pltpu.make_async_copy(dst_hbm.at[pl.ds(0, EB)], idx_d[ip],
                                  sem[ip]).wait()
            pltpu.make_async_copy(m_hbm.at[pl.ds(0, EB)], gs[ip],
                                  sem[ip]).wait()
            scatter(AGG_HALF, ip, gs[ip])

    issue1(0, 0)

    def blk1(bi2, carry):
        for p in range(2):
            bi = bi2 * 2 + p
            issue1(bi + 1, 1 - p)
            work1(bi, p)
        return carry

    lax.fori_loop(0, NB_W // 2, blk1, 0)
    dump_agg(AGG_HALF)


import functools


@functools.lru_cache(maxsize=None)
def _sc_mesh():
    return plsc.VectorSubcoreMesh(core_axis_name="c", subcore_axis_name="s",
                                  num_cores=NC, num_subcores=NS)


@functools.lru_cache(maxsize=None)
def _dd_kernel():
    return pl.kernel(
        _dd_body,
        out_type=jax.ShapeDtypeStruct((E, H), F32),
        mesh=_sc_mesh(),
        scratch_types=[
            pltpu.VMEM((EB,), jnp.int32),
            pltpu.VMEM((EB,), jnp.int32),
            pltpu.VMEM((EB, H), F32),
            pltpu.VMEM((EB, H), F32),
            pltpu.VMEM((EB,), jnp.int32),
            pltpu.VMEM((EB,), jnp.int32),
            pltpu.VMEM((EB, H), F32),
            pltpu.VMEM((EB, H), F32),
            pltpu.VMEM((EB, H), F32),
            pltpu.SemaphoreType.DMA,
            pltpu.SemaphoreType.DMA,
        ],
    )


@functools.lru_cache(maxsize=None)
def _edge_kernel():
    return pl.kernel(
        _edge_body,
        out_type=(jax.ShapeDtypeStruct((NC, NPAD, H), F32),
                  jax.ShapeDtypeStruct((E, H), F32)),
        mesh=_sc_mesh(),
        scratch_types=[
            pltpu.VMEM((EB,), jnp.int32),
            pltpu.VMEM((EB,), jnp.int32),
            pltpu.VMEM((EB, H), F32),
            pltpu.VMEM((EB, H), F32),
            pltpu.VMEM((EB,), jnp.int32),
            pltpu.VMEM((EB,), jnp.int32),
            pltpu.VMEM((EB, H), F32),
            pltpu.VMEM((EB, H), F32),
            pltpu.VMEM((EB, H), F32),
            pltpu.VMEM_SHARED((AGG_ROWS, H), F32),
            pltpu.SemaphoreType.DMA,
            pltpu.SemaphoreType.DMA,
        ],
    )


def _dd_call(fpad, src, dst):
    return _dd_kernel()(fpad, src, dst)


def _edge_call(xs, xd, ffw, src, dst):
    aggp, _ = _edge_kernel()(xs, xd, ffw, src, dst)
    return aggp


def kernel(t, pos, h, l, W_emb, b_emb, W_t, W_lin, W_vin, We_s, We_d, We_f,
           be, Wn, bn, ln_g, ln_b, W_vout, W_lout, node_index,
           edge_node_index):
    nkey = jax.random.key(1)
    eps_l = jax.random.normal(jax.random.fold_in(nkey, 0), l.shape, dtype=F32)
    eps_r = jax.random.normal(jax.random.fold_in(nkey, 1), pos.shape, dtype=F32)
    eps_v = jax.random.normal(jax.random.fold_in(nkey, 2), pos.shape, dtype=F32)

    padc = lambda a: jnp.pad(a, ((0, 0), (0, H - a.shape[1])))
    padr = lambda a: jnp.pad(a, ((0, H - a.shape[0]), (0, 0)))
    pos_p = padc(pos)
    epsr_p = padc(eps_r)
    epsv_p = padc(eps_v)
    h_p = padc(h)
    wemb_p = padr(W_emb)
    wlin_p = padr(W_lin)
    wvin_p = padr(W_vin)
    wvout_p = padc(W_vout)
    wlout_p = padc(W_lout)
    l9p = padc(l.reshape(G, 9))
    epsl9p = padc(eps_l.reshape(G, 9))
    t_mat = jnp.broadcast_to(t[:, None], (G, H))
    nid_b = jnp.broadcast_to(node_index[:, None], (N, H))
    bemb_m = jnp.broadcast_to(b_emb[None, :], (G, H))
    src = edge_node_index[0]
    dst = edge_node_index[1]

    nblk = pl.BlockSpec((NB, H), lambda i: (i, 0))
    gblk = pl.BlockSpec((G, H), lambda i: (0, 0))
    wblk = pl.BlockSpec((H, H), lambda i: (0, 0))

    cnt, sv, su, fpad = pl.pallas_call(
        _prep_a,
        grid=(N // NB,),
        in_specs=[nblk, gblk, nblk, nblk, nblk],
        out_specs=[gblk, gblk, gblk, nblk],
        out_shape=[jax.ShapeDtypeStruct((G, H), F32)] * 3
        + [jax.ShapeDtypeStruct((N, H), F32)],
    )(nid_b, t_mat, pos_p, epsr_p, epsv_p)

    x0, tv = pl.pallas_call(
        _prep_b,
        grid=(N // NB,),
        in_specs=[nblk, gblk, nblk, nblk, nblk, nblk,
                  wblk, wblk, wblk, wblk, gblk, gblk, gblk,
                  gblk, gblk, gblk],
        out_specs=[nblk, nblk],
        out_shape=[jax.ShapeDtypeStruct((N, H), F32)] * 2,
    )(nid_b, t_mat, pos_p, epsr_p, epsv_p, h_p, wemb_p, W_t, wlin_p,
      wvin_p, l9p, epsl9p, bemb_m, cnt, sv, su)

    dd = _dd_call(fpad, src, dst)

    eblk = pl.BlockSpec((BE, H), lambda i: (i, 0))
    ffw = pl.pallas_call(
        _fourier,
        grid=(E // BE,),
        in_specs=[eblk,
                  pl.BlockSpec((NL, 768, H), lambda i: (0, 0, 0))],
        out_specs=[eblk, eblk, eblk, eblk],
        out_shape=[jax.ShapeDtypeStruct((E, H), F32)] * NL,
    )(dd, We_f.astype(jnp.bfloat16))

    n2blk = pl.BlockSpec((NB2, H), lambda i: (i, 0))
    rowblk = pl.BlockSpec((8, H), lambda i: (0, 0))
    a0blk = pl.BlockSpec((1, NB2, H), lambda i: (0, i, 0))
    a1blk = pl.BlockSpec((1, NB2, H), lambda i: (1, i, 0))
    be_b = [jnp.broadcast_to(be[i][None, :], (8, H)) for i in range(NL)]
    bn_b = [jnp.broadcast_to(bn[i][None, :], (8, H)) for i in range(NL)]
    g_b = [jnp.broadcast_to(ln_g[i][None, :], (8, H)) for i in range(NL)]
    b_b = [jnp.broadcast_to(ln_b[i][None, :], (8, H)) for i in range(NL)]

    x = x0
    xs, xd = pl.pallas_call(
        _node_mm,
        grid=(N // NB2,),
        in_specs=[n2blk, wblk, wblk, rowblk],
        out_specs=[n2blk, n2blk],
        out_shape=[jax.ShapeDtypeStruct((N, H), F32)] * 2,
    )(x, We_s[0], We_d[0], be_b[0])

    for i in range(NL):
        aggp = _edge_call(xs, xd, ffw[i], src, dst)
        if i < NL - 1:
            x, xs, xd = pl.pallas_call(
                _update_mm,
                grid=(N // NB2,),
                in_specs=[n2blk, a0blk, a1blk, wblk, wblk, rowblk, rowblk,
                          rowblk, wblk, wblk, rowblk],
                out_specs=[n2blk, n2blk, n2blk],
                out_shape=[jax.ShapeDtypeStruct((N, H), F32)] * 3,
            )(x, aggp, aggp, Wn[i, :H, :], Wn[i, H:, :], bn_b[i], g_b[i],
              b_b[i], We_s[i + 1], We_d[i + 1], be_b[i + 1])
        else:
            x = pl.pallas_call(
                _update,
                grid=(N // NB2,),
                in_specs=[n2blk, a0blk, a1blk, wblk, wblk, rowblk, rowblk,
                          rowblk],
                out_specs=n2blk,
                out_shape=jax.ShapeDtypeStruct((N, H), F32),
            )(x, aggp, aggp, Wn[i, :H, :], Wn[i, H:, :], bn_b[i], g_b[i],
              b_b[i])

    pooled, sy = pl.pallas_call(
        _head_a,
        grid=(N // NB,),
        in_specs=[nblk, nblk, wblk],
        out_specs=[gblk, gblk],
        out_shape=[jax.ShapeDtypeStruct((G, H), F32)] * 2,
    )(x, nid_b, wvout_p)

    loss = pl.pallas_call(
        _head_b,
        grid=(N // NB,),
        in_specs=[nblk, nblk, nblk, wblk, gblk, gblk, gblk, wblk, gblk],
        out_specs=pl.BlockSpec((1, 1), lambda i: (0, 0)),
        out_shape=jax.ShapeDtypeStruct((1, 1), F32),
    )(x, nid_b, tv, wvout_p, cnt, sy, pooled, wlout_p, epsl9p)

    return loss[0, 0]
